# Initial kernel scaffold; baseline (speedup 1.0000x reference)
import functools

import jax
import jax.numpy as jnp
from jax import lax
from jax.experimental import pallas as pl
from jax.experimental.pallas import tpu as pltpu
from jax.experimental.pallas import tpu_sc as plsc

NC = 2
NS = 16
LN = 16
NW = NC * NS
CHUNK = 80


def _norm_body(x_ref, nh_ref):
    x = x_ref[...]
    s = jnp.sum(x * x, axis=1, keepdims=True)
    nh_ref[...] = x / jnp.maximum(jnp.sqrt(s), 1e-12)


def _tc_normalize(x):
    n, d = x.shape
    blk = 1000
    return pl.pallas_call(
        _norm_body,
        grid=(n // blk,),
        in_specs=[pl.BlockSpec((blk, d), lambda i: (i, 0))],
        out_specs=pl.BlockSpec((blk, d), lambda i: (i, 0)),
        out_shape=jax.ShapeDtypeStruct((n, d), jnp.float32),
    )(x)


def _combine_body(parts_ref, h_ref, nh_ref):
    h = jnp.maximum(parts_ref[0] + parts_ref[1], 0.0)
    h_ref[...] = h
    s = jnp.sum(h * h, axis=1, keepdims=True)
    nh_ref[...] = h / jnp.maximum(jnp.sqrt(s), 1e-12)


def _tc_combine(parts):
    _, n, d = parts.shape
    blk = 1000
    return pl.pallas_call(
        _combine_body,
        grid=(n // blk,),
        in_specs=[pl.BlockSpec((2, blk, d), lambda i: (0, i, 0))],
        out_specs=(pl.BlockSpec((blk, d), lambda i: (i, 0)),
                   pl.BlockSpec((blk, d), lambda i: (i, 0))),
        out_shape=(jax.ShapeDtypeStruct((n, d), jnp.float32),
                   jax.ShapeDtypeStruct((n, d), jnp.float32)),
    )(parts)


def _sc_mesh():
    return plsc.VectorSubcoreMesh(
        core_axis_name="c", subcore_axis_name="s",
        num_cores=NC, num_subcores=NS)


@functools.lru_cache(maxsize=None)
def _make_sc_edge_weights(n, e, d):
    epw = e // NW
    nchunk = epw // CHUNK
    kd = d // LN

    @functools.partial(
        pl.kernel,
        out_type=(jax.ShapeDtypeStruct((e,), jnp.float32),
                  jax.ShapeDtypeStruct((NC, n), jnp.float32)),
        mesh=_sc_mesh(),
        scratch_types=[
            pltpu.VMEM((CHUNK,), jnp.int32),
            pltpu.VMEM((CHUNK,), jnp.int32),
            pltpu.VMEM((CHUNK, d), jnp.float32),
            pltpu.VMEM((CHUNK, d), jnp.float32),
            pltpu.VMEM((LN * LN,), jnp.float32),
            pltpu.VMEM((CHUNK,), jnp.float32),
            pltpu.VMEM((LN,), jnp.float32),
            pltpu.VMEM((n,), jnp.float32),
            pltpu.VMEM_SHARED((n,), jnp.float32),
            pltpu.SemaphoreType.DMA,
            pltpu.SemaphoreType.DMA,
        ],
    )
    def kern(nh_hbm, src_hbm, dst_hbm, beta_hbm, w_hbm, denom_hbm,
             src_v, dst_v, srows, drows, accv, dots, betav, nbuf, denom_sh,
             sem0, sem1):
        cid = lax.axis_index("c")
        sid = lax.axis_index("s")
        wid = cid * NS + sid

        @pl.when(sid == 0)
        def _zero_denom():
            def zb(i, carry):
                nbuf[pl.ds(i * LN, LN)] = jnp.zeros((LN,), jnp.float32)
                return carry
            lax.fori_loop(0, n // LN, zb, 0)
            pltpu.sync_copy(nbuf, denom_sh)

        pltpu.sync_copy(beta_hbm, betav)
        plsc.subcore_barrier()

        bvec = betav[...]
        absb = jnp.abs(bvec)
        base0 = wid * epw
        lane = lax.broadcasted_iota(jnp.int32, (LN,), 0)

        def chunk(j, carry):
            base = base0 + j * CHUNK
            pltpu.sync_copy(src_hbm.at[pl.ds(base, CHUNK)], src_v)
            pltpu.sync_copy(dst_hbm.at[pl.ds(base, CHUNK)], dst_v)
            cp0 = pltpu.async_copy(nh_hbm.at[src_v], srows, sem0)
            cp1 = pltpu.async_copy(nh_hbm.at[dst_v], drows, sem1)
            cp0.wait()
            cp1.wait()

            def grp(g, c2):
                for i in range(LN):
                    ei = g * LN + i
                    acc = srows[ei, pl.ds(0, LN)] * drows[ei, pl.ds(0, LN)]
                    for k in range(1, kd):
                        acc = acc + (srows[ei, pl.ds(k * LN, LN)]
                                     * drows[ei, pl.ds(k * LN, LN)])
                    accv[pl.ds(i * LN, LN)] = acc
                ii = lane * LN
                tot = plsc.load_gather(accv, [ii])
                for l in range(1, LN):
                    tot = tot + plsc.load_gather(accv, [ii + l])
                dots[pl.ds(g * LN, LN)] = tot
                return c2
            lax.fori_loop(0, CHUNK // LN, grp, 0)

            def wv(g, c2):
                t = dots[pl.ds(g * LN, LN)]
                dots[pl.ds(g * LN, LN)] = jnp.exp(bvec * t - absb)
                return c2
            lax.fori_loop(0, CHUNK // LN, wv, 0)

            pltpu.sync_copy(dots, w_hbm.at[pl.ds(base, CHUNK)])
            pltpu.sync_copy(dots, denom_sh.at[dst_v], add=True)
            return carry
        lax.fori_loop(0, nchunk, chunk, 0)

        plsc.subcore_barrier()

        @pl.when(sid == 0)
        def _dump():
            pltpu.sync_copy(denom_sh, nbuf)
            pltpu.sync_copy(nbuf, denom_hbm.at[cid])

    return kern


@functools.lru_cache(maxsize=None)
def _make_sc_aggregate(n, e, d):
    epw = e // NW
    nchunk = epw // CHUNK
    kd = d // LN
    nzc = n // CHUNK

    @functools.partial(
        pl.kernel,
        out_type=jax.ShapeDtypeStruct((NC, n, d), jnp.float32),
        mesh=_sc_mesh(),
        scratch_types=[
            pltpu.VMEM((CHUNK,), jnp.int32),
            pltpu.VMEM((CHUNK,), jnp.int32),
            pltpu.VMEM((CHUNK, d), jnp.float32),
            pltpu.VMEM((CHUNK,), jnp.float32),
            pltpu.VMEM((CHUNK,), jnp.float32),
            pltpu.VMEM((CHUNK,), jnp.float32),
            pltpu.VMEM_SHARED((n, d), jnp.float32),
            pltpu.SemaphoreType.DMA,
            pltpu.SemaphoreType.DMA,
            pltpu.SemaphoreType.DMA,
        ],
    )
    def kern(h_hbm, src_hbm, dst_hbm, w_hbm, d0_hbm, d1_hbm, out_hbm,
             src_v, dst_v, rows, wv, d0v, d1v, out_sh, sem0, sem1, sem2):
        cid = lax.axis_index("c")
        sid = lax.axis_index("s")
        wid = cid * NS + sid

        def zr(i, carry):
            r = i // kd
            c2 = i % kd
            rows[r, pl.ds(c2 * LN, LN)] = jnp.zeros((LN,), jnp.float32)
            return carry
        lax.fori_loop(0, CHUNK * kd, zr, 0)
        for t in range((nzc + NS - 1) // NS):
            k = sid + t * NS

            @pl.when(k < nzc)
            def _zc():
                pltpu.sync_copy(rows, out_sh.at[pl.ds(k * CHUNK, CHUNK)])
        plsc.subcore_barrier()

        base0 = wid * epw

        def chunk(j, carry):
            base = base0 + j * CHUNK
            pltpu.sync_copy(src_hbm.at[pl.ds(base, CHUNK)], src_v)
            pltpu.sync_copy(dst_hbm.at[pl.ds(base, CHUNK)], dst_v)
            pltpu.sync_copy(w_hbm.at[pl.ds(base, CHUNK)], wv)
            cp0 = pltpu.async_copy(h_hbm.at[src_v], rows, sem0)
            cp1 = pltpu.async_copy(d0_hbm.at[dst_v], d0v, sem1)
            cp2 = pltpu.async_copy(d1_hbm.at[dst_v], d1v, sem2)
            cp0.wait()
            cp1.wait()
            cp2.wait()

            def pv(g, c2):
                sl = pl.ds(g * LN, LN)
                wv[sl] = wv[sl] / (d0v[sl] + d1v[sl])
                return c2
            lax.fori_loop(0, CHUNK // LN, pv, 0)

            def scale(ei, c2):
                p = jnp.full((LN,), wv[ei], jnp.float32)
                for k in range(kd):
                    sl = pl.ds(k * LN, LN)
                    rows[ei, sl] = rows[ei, sl] * p
                return c2
            lax.fori_loop(0, CHUNK, scale, 0)

            pltpu.sync_copy(rows, out_sh.at[dst_v], add=True)
            return carry
        lax.fori_loop(0, nchunk, chunk, 0)

        plsc.subcore_barrier()
        for t in range((nzc + NS - 1) // NS):
            k = sid + t * NS

            @pl.when(k < nzc)
            def _dump():
                pltpu.sync_copy(out_sh.at[pl.ds(k * CHUNK, CHUNK)], rows)
                pltpu.sync_copy(rows, out_hbm.at[cid, pl.ds(k * CHUNK, CHUNK)])

    return kern


def kernel(x, edge_index, beta):
    n, d = x.shape
    e = edge_index.shape[1]
    src = edge_index[0]
    dst = edge_index[1]
    sc_a = _make_sc_edge_weights(n, e, d)
    sc_b = _make_sc_aggregate(n, e, d)
    h = x
    nh = _tc_normalize(x)
    for i in range(beta.shape[0]):
        beta16 = jnp.full((LN,), beta[i], jnp.float32)
        w, denom = sc_a(nh, src, dst, beta16)
        parts = sc_b(h, src, dst, w, denom[0], denom[1])
        h, nh = _tc_combine(parts)
    return h


# speedup vs baseline: 6.5350x; 6.5350x over previous
import functools

import jax
import jax.numpy as jnp
from jax import lax
from jax.experimental import pallas as pl
from jax.experimental.pallas import tpu as pltpu
from jax.experimental.pallas import tpu_sc as plsc

NC = 2
NS = 16
LN = 16
NW = NC * NS
CHUNK = 80


def _norm_body(x_ref, nh_ref):
    x = x_ref[...]
    s = jnp.sum(x * x, axis=1, keepdims=True)
    nh_ref[...] = x / jnp.maximum(jnp.sqrt(s), 1e-12)


def _tc_normalize(x):
    n, d = x.shape
    blk = 1000
    return pl.pallas_call(
        _norm_body,
        grid=(n // blk,),
        in_specs=[pl.BlockSpec((blk, d), lambda i: (i, 0))],
        out_specs=pl.BlockSpec((blk, d), lambda i: (i, 0)),
        out_shape=jax.ShapeDtypeStruct((n, d), jnp.float32),
    )(x)


def _combine_body(parts_ref, h_ref, nh_ref):
    h = jnp.maximum(parts_ref[0] + parts_ref[1], 0.0)
    h_ref[...] = h
    s = jnp.sum(h * h, axis=1, keepdims=True)
    nh_ref[...] = h / jnp.maximum(jnp.sqrt(s), 1e-12)


def _tc_combine(parts):
    _, n, d = parts.shape
    blk = 1000
    return pl.pallas_call(
        _combine_body,
        grid=(n // blk,),
        in_specs=[pl.BlockSpec((2, blk, d), lambda i: (0, i, 0))],
        out_specs=(pl.BlockSpec((blk, d), lambda i: (i, 0)),
                   pl.BlockSpec((blk, d), lambda i: (i, 0))),
        out_shape=(jax.ShapeDtypeStruct((n, d), jnp.float32),
                   jax.ShapeDtypeStruct((n, d), jnp.float32)),
    )(parts)


def _sc_mesh():
    return plsc.VectorSubcoreMesh(
        core_axis_name="c", subcore_axis_name="s",
        num_cores=NC, num_subcores=NS)


@functools.lru_cache(maxsize=None)
def _make_sc_edge_weights(n, e, d):
    epw = e // NW
    nchunk = epw // CHUNK
    kd = d // LN

    @functools.partial(
        pl.kernel,
        out_type=(jax.ShapeDtypeStruct((e,), jnp.float32),
                  jax.ShapeDtypeStruct((NC, n), jnp.float32)),
        mesh=_sc_mesh(),
        compiler_params=pltpu.CompilerParams(needs_layout_passes=False),
        scratch_types=[
            pltpu.VMEM((CHUNK,), jnp.int32),
            pltpu.VMEM((CHUNK,), jnp.int32),
            pltpu.VMEM((CHUNK, d), jnp.float32),
            pltpu.VMEM((CHUNK, d), jnp.float32),
            pltpu.VMEM((LN * LN,), jnp.float32),
            pltpu.VMEM((CHUNK,), jnp.float32),
            pltpu.VMEM((LN,), jnp.float32),
            pltpu.VMEM((n,), jnp.float32),
            pltpu.VMEM_SHARED((n,), jnp.float32),
            pltpu.SemaphoreType.DMA,
            pltpu.SemaphoreType.DMA,
        ],
    )
    def kern(nh_hbm, src_hbm, dst_hbm, beta_hbm, w_hbm, denom_hbm,
             src_v, dst_v, srows, drows, accv, dots, betav, nbuf, denom_sh,
             sem0, sem1):
        cid = lax.axis_index("c")
        sid = lax.axis_index("s")
        wid = cid * NS + sid

        @pl.when(sid == 0)
        def _zero_denom():
            def zb(i, carry):
                nbuf[pl.ds(i * LN, LN)] = jnp.zeros((LN,), jnp.float32)
                return carry
            lax.fori_loop(0, n // LN, zb, 0)
            pltpu.sync_copy(nbuf, denom_sh)

        pltpu.sync_copy(beta_hbm, betav)
        plsc.subcore_barrier()

        bvec = betav[...]
        absb = jnp.abs(bvec)
        base0 = wid * epw
        lane = lax.broadcasted_iota(jnp.int32, (LN,), 0)

        def chunk(j, carry):
            base = base0 + j * CHUNK
            pltpu.sync_copy(src_hbm.at[pl.ds(base, CHUNK)], src_v)
            pltpu.sync_copy(dst_hbm.at[pl.ds(base, CHUNK)], dst_v)
            cp0 = pltpu.async_copy(nh_hbm.at[src_v], srows, sem0)
            cp1 = pltpu.async_copy(nh_hbm.at[dst_v], drows, sem1)
            cp0.wait()
            cp1.wait()

            def grp(g, c2):
                for i in range(LN):
                    ei = g * LN + i
                    acc = srows[ei, pl.ds(0, LN)] * drows[ei, pl.ds(0, LN)]
                    for k in range(1, kd):
                        acc = acc + (srows[ei, pl.ds(k * LN, LN)]
                                     * drows[ei, pl.ds(k * LN, LN)])
                    accv[pl.ds(i * LN, LN)] = acc
                ii = lane * LN
                tot = plsc.load_gather(accv, [ii])
                for l in range(1, LN):
                    tot = tot + plsc.load_gather(accv, [ii + l])
                dots[pl.ds(g * LN, LN)] = tot
                return c2
            lax.fori_loop(0, CHUNK // LN, grp, 0)

            def wv(g, c2):
                t = dots[pl.ds(g * LN, LN)]
                dots[pl.ds(g * LN, LN)] = jnp.exp(bvec * t - absb)
                return c2
            lax.fori_loop(0, CHUNK // LN, wv, 0)

            pltpu.sync_copy(dots, w_hbm.at[pl.ds(base, CHUNK)])
            pltpu.sync_copy(dots, denom_sh.at[dst_v], add=True)
            return carry
        lax.fori_loop(0, nchunk, chunk, 0)

        plsc.subcore_barrier()

        @pl.when(sid == 0)
        def _dump():
            pltpu.sync_copy(denom_sh, nbuf)
            pltpu.sync_copy(nbuf, denom_hbm.at[cid])

    return kern


@functools.lru_cache(maxsize=None)
def _make_sc_aggregate(n, e, d):
    epw = e // NW
    nchunk = epw // CHUNK
    kd = d // LN
    nzc = n // CHUNK

    @functools.partial(
        pl.kernel,
        out_type=jax.ShapeDtypeStruct((NC, n, d), jnp.float32),
        mesh=_sc_mesh(),
        compiler_params=pltpu.CompilerParams(needs_layout_passes=False),
        scratch_types=[
            pltpu.VMEM((CHUNK,), jnp.int32),
            pltpu.VMEM((CHUNK,), jnp.int32),
            pltpu.VMEM((CHUNK, d), jnp.float32),
            pltpu.VMEM((CHUNK,), jnp.float32),
            pltpu.VMEM((CHUNK,), jnp.float32),
            pltpu.VMEM((CHUNK,), jnp.float32),
            pltpu.VMEM_SHARED((n, d), jnp.float32),
            pltpu.SemaphoreType.DMA,
            pltpu.SemaphoreType.DMA,
            pltpu.SemaphoreType.DMA,
        ],
    )
    def kern(h_hbm, src_hbm, dst_hbm, w_hbm, d0_hbm, d1_hbm, out_hbm,
             src_v, dst_v, rows, wv, d0v, d1v, out_sh, sem0, sem1, sem2):
        cid = lax.axis_index("c")
        sid = lax.axis_index("s")
        wid = cid * NS + sid

        def zr(i, carry):
            r = i // kd
            c2 = i % kd
            rows[r, pl.ds(c2 * LN, LN)] = jnp.zeros((LN,), jnp.float32)
            return carry
        lax.fori_loop(0, CHUNK * kd, zr, 0)
        for t in range((nzc + NS - 1) // NS):
            k = sid + t * NS

            @pl.when(k < nzc)
            def _zc():
                pltpu.sync_copy(rows, out_sh.at[pl.ds(k * CHUNK, CHUNK)])
        plsc.subcore_barrier()

        base0 = wid * epw

        def chunk(j, carry):
            base = base0 + j * CHUNK
            pltpu.sync_copy(src_hbm.at[pl.ds(base, CHUNK)], src_v)
            pltpu.sync_copy(dst_hbm.at[pl.ds(base, CHUNK)], dst_v)
            pltpu.sync_copy(w_hbm.at[pl.ds(base, CHUNK)], wv)
            cp0 = pltpu.async_copy(h_hbm.at[src_v], rows, sem0)
            cp1 = pltpu.async_copy(d0_hbm.at[dst_v], d0v, sem1)
            cp2 = pltpu.async_copy(d1_hbm.at[dst_v], d1v, sem2)
            cp0.wait()
            cp1.wait()
            cp2.wait()

            def pv(g, c2):
                sl = pl.ds(g * LN, LN)
                wv[sl] = wv[sl] / (d0v[sl] + d1v[sl])
                return c2
            lax.fori_loop(0, CHUNK // LN, pv, 0)

            def scale(ei, c2):
                p = plsc.load_gather(wv, [jnp.full((LN,), ei, jnp.int32)])
                for k in range(kd):
                    sl = pl.ds(k * LN, LN)
                    rows[ei, sl] = rows[ei, sl] * p
                return c2
            lax.fori_loop(0, CHUNK, scale, 0)

            pltpu.sync_copy(rows, out_sh.at[dst_v], add=True)
            return carry
        lax.fori_loop(0, nchunk, chunk, 0)

        plsc.subcore_barrier()
        for t in range((nzc + NS - 1) // NS):
            k = sid + t * NS

            @pl.when(k < nzc)
            def _dump():
                pltpu.sync_copy(out_sh.at[pl.ds(k * CHUNK, CHUNK)], rows)
                pltpu.sync_copy(rows, out_hbm.at[cid, pl.ds(k * CHUNK, CHUNK)])

    return kern


def kernel(x, edge_index, beta):
    n, d = x.shape
    e = edge_index.shape[1]
    src = edge_index[0]
    dst = edge_index[1]
    sc_a = _make_sc_edge_weights(n, e, d)
    sc_b = _make_sc_aggregate(n, e, d)
    h = x
    nh = _tc_normalize(x)
    for i in range(beta.shape[0]):
        beta16 = jnp.full((LN,), beta[i], jnp.float32)
        w, denom = sc_a(nh, src, dst, beta16)
        parts = sc_b(h, src, dst, w, denom[0], denom[1])
        h, nh = _tc_combine(parts)
    return h


# speedup vs baseline: 7.4962x; 1.1471x over previous
import functools

import jax
import jax.numpy as jnp
from jax import lax
from jax.experimental import pallas as pl
from jax.experimental.pallas import tpu as pltpu
from jax.experimental.pallas import tpu_sc as plsc

NC = 2
NS = 16
LN = 16
NW = NC * NS
CHUNK_A = 400
CHUNK_B = 80


def _norm_body(x_ref, nh_ref):
    x = x_ref[...]
    s = jnp.sum(x * x, axis=1, keepdims=True)
    nh_ref[...] = x / jnp.maximum(jnp.sqrt(s), 1e-12)


def _tc_normalize(x):
    n, d = x.shape
    blk = 1000
    return pl.pallas_call(
        _norm_body,
        grid=(n // blk,),
        in_specs=[pl.BlockSpec((blk, d), lambda i: (i, 0))],
        out_specs=pl.BlockSpec((blk, d), lambda i: (i, 0)),
        out_shape=jax.ShapeDtypeStruct((n, d), jnp.float32),
    )(x)


def _combine_body(parts_ref, h_ref, nh_ref):
    h = jnp.maximum(parts_ref[0] + parts_ref[1], 0.0)
    h_ref[...] = h
    s = jnp.sum(h * h, axis=1, keepdims=True)
    nh_ref[...] = h / jnp.maximum(jnp.sqrt(s), 1e-12)


def _tc_combine(parts):
    _, n, d = parts.shape
    blk = 1000
    return pl.pallas_call(
        _combine_body,
        grid=(n // blk,),
        in_specs=[pl.BlockSpec((2, blk, d), lambda i: (0, i, 0))],
        out_specs=(pl.BlockSpec((blk, d), lambda i: (i, 0)),
                   pl.BlockSpec((blk, d), lambda i: (i, 0))),
        out_shape=(jax.ShapeDtypeStruct((n, d), jnp.float32),
                   jax.ShapeDtypeStruct((n, d), jnp.float32)),
    )(parts)


def _sc_mesh():
    return plsc.VectorSubcoreMesh(
        core_axis_name="c", subcore_axis_name="s",
        num_cores=NC, num_subcores=NS)


@functools.lru_cache(maxsize=None)
def _make_sc_edge_weights(n, e, d):
    epw = e // NW
    nchunk = epw // CHUNK_A
    kd = d // LN

    @functools.partial(
        pl.kernel,
        out_type=(jax.ShapeDtypeStruct((e,), jnp.float32),
                  jax.ShapeDtypeStruct((NC, n), jnp.float32)),
        mesh=_sc_mesh(),
        compiler_params=pltpu.CompilerParams(needs_layout_passes=False),
        scratch_types=[
            pltpu.VMEM((CHUNK_A,), jnp.int32),
            pltpu.VMEM((CHUNK_A,), jnp.int32),
            pltpu.VMEM((CHUNK_A, d), jnp.float32),
            pltpu.VMEM((CHUNK_A, d), jnp.float32),
            pltpu.VMEM((LN * LN,), jnp.float32),
            pltpu.VMEM((CHUNK_A,), jnp.float32),
            pltpu.VMEM((LN,), jnp.float32),
            pltpu.VMEM((n,), jnp.float32),
            pltpu.VMEM_SHARED((n,), jnp.float32),
            pltpu.SemaphoreType.DMA,
            pltpu.SemaphoreType.DMA,
        ],
    )
    def kern(nh_hbm, src_hbm, dst_hbm, beta_hbm, w_hbm, denom_hbm,
             src_v, dst_v, srows, drows, accv, dots, betav, nbuf, denom_sh,
             sem0, sem1):
        cid = lax.axis_index("c")
        sid = lax.axis_index("s")
        wid = cid * NS + sid

        @pl.when(sid == 0)
        def _zero_denom():
            def zb(i, carry):
                nbuf[pl.ds(i * LN, LN)] = jnp.zeros((LN,), jnp.float32)
                return carry
            lax.fori_loop(0, n // LN, zb, 0)
            pltpu.sync_copy(nbuf, denom_sh)

        pltpu.sync_copy(beta_hbm, betav)
        plsc.subcore_barrier()

        bvec = betav[...]
        absb = jnp.abs(bvec)
        base0 = wid * epw
        lane = lax.broadcasted_iota(jnp.int32, (LN,), 0)

        def chunk(j, carry):
            base = base0 + j * CHUNK_A
            pltpu.sync_copy(src_hbm.at[pl.ds(base, CHUNK_A)], src_v)
            pltpu.sync_copy(dst_hbm.at[pl.ds(base, CHUNK_A)], dst_v)
            cp0 = pltpu.async_copy(nh_hbm.at[src_v], srows, sem0)
            cp1 = pltpu.async_copy(nh_hbm.at[dst_v], drows, sem1)
            cp0.wait()
            cp1.wait()

            def grp(g, c2):
                for i in range(LN):
                    ei = g * LN + i
                    acc = srows[ei, pl.ds(0, LN)] * drows[ei, pl.ds(0, LN)]
                    for k in range(1, kd):
                        acc = acc + (srows[ei, pl.ds(k * LN, LN)]
                                     * drows[ei, pl.ds(k * LN, LN)])
                    accv[pl.ds(i * LN, LN)] = acc
                ii = lane * LN
                tot = plsc.load_gather(accv, [ii])
                for l in range(1, LN):
                    tot = tot + plsc.load_gather(accv, [ii + l])
                dots[pl.ds(g * LN, LN)] = tot
                return c2
            lax.fori_loop(0, CHUNK_A // LN, grp, 0)

            def wv(g, c2):
                t = dots[pl.ds(g * LN, LN)]
                dots[pl.ds(g * LN, LN)] = jnp.exp(bvec * t - absb)
                return c2
            lax.fori_loop(0, CHUNK_A // LN, wv, 0)

            pltpu.sync_copy(dots, w_hbm.at[pl.ds(base, CHUNK_A)])
            pltpu.sync_copy(dots, denom_sh.at[dst_v], add=True)
            return carry
        lax.fori_loop(0, nchunk, chunk, 0)

        plsc.subcore_barrier()

        @pl.when(sid == 0)
        def _dump():
            pltpu.sync_copy(denom_sh, nbuf)
            pltpu.sync_copy(nbuf, denom_hbm.at[cid])

    return kern


@functools.lru_cache(maxsize=None)
def _make_sc_aggregate(n, e, d):
    epw = e // NW
    nchunk = epw // CHUNK_B
    kd = d // LN
    nzc = n // CHUNK_B

    @functools.partial(
        pl.kernel,
        out_type=jax.ShapeDtypeStruct((NC, n, d), jnp.float32),
        mesh=_sc_mesh(),
        compiler_params=pltpu.CompilerParams(needs_layout_passes=False),
        scratch_types=[
            pltpu.VMEM((CHUNK_B,), jnp.int32),
            pltpu.VMEM((CHUNK_B,), jnp.int32),
            pltpu.VMEM((CHUNK_B, d), jnp.float32),
            pltpu.VMEM((CHUNK_B,), jnp.float32),
            pltpu.VMEM((CHUNK_B,), jnp.float32),
            pltpu.VMEM((CHUNK_B,), jnp.float32),
            pltpu.VMEM_SHARED((n, d), jnp.float32),
            pltpu.SemaphoreType.DMA,
            pltpu.SemaphoreType.DMA,
            pltpu.SemaphoreType.DMA,
        ],
    )
    def kern(h_hbm, src_hbm, dst_hbm, w_hbm, d0_hbm, d1_hbm, out_hbm,
             src_v, dst_v, rows, wv, d0v, d1v, out_sh, sem0, sem1, sem2):
        cid = lax.axis_index("c")
        sid = lax.axis_index("s")
        wid = cid * NS + sid

        def zr(i, carry):
            r = i // kd
            c2 = i % kd
            rows[r, pl.ds(c2 * LN, LN)] = jnp.zeros((LN,), jnp.float32)
            return carry
        lax.fori_loop(0, CHUNK_B * kd, zr, 0)
        for t in range((nzc + NS - 1) // NS):
            k = sid + t * NS

            @pl.when(k < nzc)
            def _zc():
                pltpu.sync_copy(rows, out_sh.at[pl.ds(k * CHUNK_B, CHUNK_B)])
        plsc.subcore_barrier()

        base0 = wid * epw

        def chunk(j, carry):
            base = base0 + j * CHUNK_B
            pltpu.sync_copy(src_hbm.at[pl.ds(base, CHUNK_B)], src_v)
            pltpu.sync_copy(dst_hbm.at[pl.ds(base, CHUNK_B)], dst_v)
            pltpu.sync_copy(w_hbm.at[pl.ds(base, CHUNK_B)], wv)
            cp0 = pltpu.async_copy(h_hbm.at[src_v], rows, sem0)
            cp1 = pltpu.async_copy(d0_hbm.at[dst_v], d0v, sem1)
            cp2 = pltpu.async_copy(d1_hbm.at[dst_v], d1v, sem2)
            cp0.wait()
            cp1.wait()
            cp2.wait()

            def pv(g, c2):
                sl = pl.ds(g * LN, LN)
                wv[sl] = wv[sl] / (d0v[sl] + d1v[sl])
                return c2
            lax.fori_loop(0, CHUNK_B // LN, pv, 0)

            def scale(ei, c2):
                p = plsc.load_gather(wv, [jnp.full((LN,), ei, jnp.int32)])
                for k in range(kd):
                    sl = pl.ds(k * LN, LN)
                    rows[ei, sl] = rows[ei, sl] * p
                return c2
            lax.fori_loop(0, CHUNK_B, scale, 0)

            pltpu.sync_copy(rows, out_sh.at[dst_v], add=True)
            return carry
        lax.fori_loop(0, nchunk, chunk, 0)

        plsc.subcore_barrier()
        for t in range((nzc + NS - 1) // NS):
            k = sid + t * NS

            @pl.when(k < nzc)
            def _dump():
                pltpu.sync_copy(out_sh.at[pl.ds(k * CHUNK_B, CHUNK_B)], rows)
                pltpu.sync_copy(rows, out_hbm.at[cid, pl.ds(k * CHUNK_B, CHUNK_B)])

    return kern


def kernel(x, edge_index, beta):
    n, d = x.shape
    e = edge_index.shape[1]
    src = edge_index[0]
    dst = edge_index[1]
    sc_a = _make_sc_edge_weights(n, e, d)
    sc_b = _make_sc_aggregate(n, e, d)
    h = x
    nh = _tc_normalize(x)
    for i in range(beta.shape[0]):
        beta16 = jnp.full((LN,), beta[i], jnp.float32)
        w, denom = sc_a(nh, src, dst, beta16)
        parts = sc_b(h, src, dst, w, denom[0], denom[1])
        h, nh = _tc_combine(parts)
    return h


# speedup vs baseline: 8.8351x; 1.1786x over previous
import functools

import jax
import jax.numpy as jnp
from jax import lax
from jax.experimental import pallas as pl
from jax.experimental.pallas import tpu as pltpu
from jax.experimental.pallas import tpu_sc as plsc

NC = 2
NS = 16
LN = 16
NW = NC * NS
CHUNK_A = 400
CHUNK_B = 80


def _norm_body(x_ref, nh_ref):
    x = x_ref[...]
    s = jnp.sum(x * x, axis=1, keepdims=True)
    nh_ref[...] = x / jnp.maximum(jnp.sqrt(s), 1e-12)


def _tc_normalize(x):
    n, d = x.shape
    blk = 1000
    return pl.pallas_call(
        _norm_body,
        grid=(n // blk,),
        in_specs=[pl.BlockSpec((blk, d), lambda i: (i, 0))],
        out_specs=pl.BlockSpec((blk, d), lambda i: (i, 0)),
        out_shape=jax.ShapeDtypeStruct((n, d), jnp.float32),
    )(x)


def _combine_body(parts_ref, h_ref, nh_ref):
    h = jnp.maximum(parts_ref[0] + parts_ref[1], 0.0)
    h_ref[...] = h
    s = jnp.sum(h * h, axis=1, keepdims=True)
    nh_ref[...] = h / jnp.maximum(jnp.sqrt(s), 1e-12)


def _tc_combine(parts):
    _, n, d = parts.shape
    blk = 1000
    return pl.pallas_call(
        _combine_body,
        grid=(n // blk,),
        in_specs=[pl.BlockSpec((2, blk, d), lambda i: (0, i, 0))],
        out_specs=(pl.BlockSpec((blk, d), lambda i: (i, 0)),
                   pl.BlockSpec((blk, d), lambda i: (i, 0))),
        out_shape=(jax.ShapeDtypeStruct((n, d), jnp.float32),
                   jax.ShapeDtypeStruct((n, d), jnp.float32)),
    )(parts)


def _sc_mesh():
    return plsc.VectorSubcoreMesh(
        core_axis_name="c", subcore_axis_name="s",
        num_cores=NC, num_subcores=NS)


@functools.lru_cache(maxsize=None)
def _make_sc_edge_weights(n, e, d):
    epw = e // NW
    nchunk = epw // CHUNK_A
    kd = d // LN

    @functools.partial(
        pl.kernel,
        out_type=(jax.ShapeDtypeStruct((e,), jnp.float32),
                  jax.ShapeDtypeStruct((NC, n), jnp.float32)),
        mesh=_sc_mesh(),
        compiler_params=pltpu.CompilerParams(needs_layout_passes=False),
        scratch_types=[
            pltpu.VMEM((CHUNK_A,), jnp.int32),
            pltpu.VMEM((CHUNK_A,), jnp.int32),
            pltpu.VMEM((CHUNK_A, d), jnp.float32),
            pltpu.VMEM((CHUNK_A, d), jnp.float32),
            pltpu.VMEM((LN * LN,), jnp.float32),
            pltpu.VMEM((CHUNK_A,), jnp.float32),
            pltpu.VMEM((LN,), jnp.float32),
            pltpu.VMEM((n,), jnp.float32),
            pltpu.VMEM_SHARED((n,), jnp.float32),
            pltpu.SemaphoreType.DMA,
            pltpu.SemaphoreType.DMA,
        ],
    )
    def kern(nh_hbm, src_hbm, dst_hbm, beta_hbm, w_hbm, denom_hbm,
             src_v, dst_v, srows, drows, accv, dots, betav, nbuf, denom_sh,
             sem0, sem1):
        cid = lax.axis_index("c")
        sid = lax.axis_index("s")
        wid = cid * NS + sid

        @pl.when(sid == 0)
        def _zero_denom():
            def zb(i, carry):
                nbuf[pl.ds(i * LN, LN)] = jnp.zeros((LN,), jnp.float32)
                return carry
            lax.fori_loop(0, n // LN, zb, 0)
            pltpu.sync_copy(nbuf, denom_sh)

        pltpu.sync_copy(beta_hbm, betav)
        plsc.subcore_barrier()

        bvec = betav[...]
        absb = jnp.abs(bvec)
        base0 = wid * epw
        lane = lax.broadcasted_iota(jnp.int32, (LN,), 0)

        def chunk(j, carry):
            base = base0 + j * CHUNK_A
            pltpu.sync_copy(src_hbm.at[pl.ds(base, CHUNK_A)], src_v)
            pltpu.sync_copy(dst_hbm.at[pl.ds(base, CHUNK_A)], dst_v)
            cp0 = pltpu.async_copy(nh_hbm.at[src_v], srows, sem0)
            cp1 = pltpu.async_copy(nh_hbm.at[dst_v], drows, sem1)
            cp0.wait()
            cp1.wait()

            def grp(g, c2):
                for i in range(LN):
                    ei = g * LN + i
                    acc = srows[ei, pl.ds(0, LN)] * drows[ei, pl.ds(0, LN)]
                    for k in range(1, kd):
                        acc = acc + (srows[ei, pl.ds(k * LN, LN)]
                                     * drows[ei, pl.ds(k * LN, LN)])
                    accv[pl.ds(i * LN, LN)] = acc
                ii = lane * LN
                tot = plsc.load_gather(accv, [ii])
                for l in range(1, LN):
                    tot = tot + plsc.load_gather(accv, [ii + l])
                dots[pl.ds(g * LN, LN)] = tot
                return c2
            lax.fori_loop(0, CHUNK_A // LN, grp, 0)

            def wv(g, c2):
                t = dots[pl.ds(g * LN, LN)]
                dots[pl.ds(g * LN, LN)] = jnp.exp(bvec * t - absb)
                return c2
            lax.fori_loop(0, CHUNK_A // LN, wv, 0)

            pltpu.sync_copy(dots, w_hbm.at[pl.ds(base, CHUNK_A)])
            pltpu.sync_copy(dots, denom_sh.at[dst_v], add=True)
            return carry
        lax.fori_loop(0, nchunk, chunk, 0)

        plsc.subcore_barrier()

        @pl.when(sid == 0)
        def _dump():
            pltpu.sync_copy(denom_sh, nbuf)
            pltpu.sync_copy(nbuf, denom_hbm.at[cid])

    return kern


@functools.lru_cache(maxsize=None)
def _make_sc_aggregate(n, e, d):
    epw = e // NW
    nchunk = epw // CHUNK_B
    kd = d // LN
    nzc = n // CHUNK_B

    @functools.partial(
        pl.kernel,
        out_type=jax.ShapeDtypeStruct((NC, n, d), jnp.float32),
        mesh=_sc_mesh(),
        compiler_params=pltpu.CompilerParams(needs_layout_passes=False),
        scratch_types=[
            pltpu.VMEM((2, CHUNK_B), jnp.int32),
            pltpu.VMEM((2, CHUNK_B), jnp.int32),
            pltpu.VMEM((CHUNK_B,), jnp.float32),
            pltpu.VMEM((CHUNK_B,), jnp.float32),
            pltpu.VMEM((CHUNK_B, d), jnp.float32),
            pltpu.VMEM((CHUNK_B, d), jnp.float32),
            pltpu.VMEM((n,), jnp.float32),
            pltpu.VMEM((n,), jnp.float32),
            pltpu.VMEM_SHARED((n, d), jnp.float32),
            pltpu.SemaphoreType.DMA,
            pltpu.SemaphoreType.DMA,
            pltpu.SemaphoreType.DMA,
            pltpu.SemaphoreType.DMA,
        ],
    )
    def kern(h_hbm, src_hbm, dst_hbm, w_hbm, d0_hbm, d1_hbm, out_hbm,
             ei0, ei1, w0, w1, rows0, rows1, d0buf, d1buf, out_sh,
             g0, g1, s0, s1):
        cid = lax.axis_index("c")
        sid = lax.axis_index("s")
        wid = cid * NS + sid
        eis = (ei0, ei1)
        wvs = (w0, w1)
        rowss = (rows0, rows1)
        gsems = (g0, g1)
        ssems = (s0, s1)

        def zr(i, carry):
            rows0[i // kd, pl.ds((i % kd) * LN, LN)] = jnp.zeros(
                (LN,), jnp.float32)
            return carry
        lax.fori_loop(0, CHUNK_B * kd, zr, 0)
        for t in range((nzc + NS - 1) // NS):
            k = sid + t * NS

            @pl.when(k < nzc)
            def _zc():
                pltpu.sync_copy(rows0, out_sh.at[pl.ds(k * CHUNK_B, CHUNK_B)])
        pltpu.sync_copy(d0_hbm, d0buf)
        pltpu.sync_copy(d1_hbm, d1buf)
        plsc.subcore_barrier()

        base0 = wid * epw

        def load_idx(b, j):
            base = base0 + j * CHUNK_B
            pltpu.sync_copy(src_hbm.at[pl.ds(base, CHUNK_B)], eis[b].at[0])
            pltpu.sync_copy(dst_hbm.at[pl.ds(base, CHUNK_B)], eis[b].at[1])
            pltpu.sync_copy(w_hbm.at[pl.ds(base, CHUNK_B)], wvs[b])

        def issue_gather(b):
            pltpu.async_copy(h_hbm.at[eis[b].at[0]], rowss[b], gsems[b])

        def wait_gather(b):
            pltpu.make_async_copy(
                h_hbm.at[eis[b].at[0]], rowss[b], gsems[b]).wait()

        def issue_scatter(b):
            pltpu.async_copy(
                rowss[b], out_sh.at[eis[b].at[1]], ssems[b], add=True)

        def wait_scatter(b):
            pltpu.make_async_copy(
                rowss[b], out_sh.at[eis[b].at[1]], ssems[b]).wait()

        def compute(b):
            ei_b, wv_b, rows_b = eis[b], wvs[b], rowss[b]

            def pv(g, c2):
                sl = pl.ds(g * LN, LN)
                dv = ei_b[1, sl]
                a0 = plsc.load_gather(d0buf, [dv])
                a1 = plsc.load_gather(d1buf, [dv])
                wv_b[sl] = wv_b[sl] / (a0 + a1)
                return c2
            lax.fori_loop(0, CHUNK_B // LN, pv, 0)

            def scale(eidx, c2):
                p = plsc.load_gather(
                    wv_b, [jnp.full((LN,), eidx, jnp.int32)])
                for k in range(kd):
                    sl = pl.ds(k * LN, LN)
                    rows_b[eidx, sl] = rows_b[eidx, sl] * p
                return c2
            lax.fori_loop(0, CHUNK_B, scale, 0)

        load_idx(0, 0)
        issue_gather(0)

        def pair(j2, carry):
            for b in (0, 1):
                j = 2 * j2 + b
                wait_gather(b)

                @pl.when(j >= 1)
                def _():
                    wait_scatter(b ^ 1)
                load_idx(b ^ 1, j + 1)
                issue_gather(b ^ 1)
                compute(b)
                issue_scatter(b)
            return carry
        lax.fori_loop(0, (nchunk - 1) // 2, pair, 0)

        wait_gather(0)
        wait_scatter(1)
        compute(0)
        pltpu.sync_copy(rows0, out_sh.at[ei0.at[1]], add=True)

        plsc.subcore_barrier()
        for t in range((nzc + NS - 1) // NS):
            k = sid + t * NS

            @pl.when(k < nzc)
            def _dump():
                pltpu.sync_copy(out_sh.at[pl.ds(k * CHUNK_B, CHUNK_B)], rows0)
                pltpu.sync_copy(rows0, out_hbm.at[cid, pl.ds(k * CHUNK_B, CHUNK_B)])

    return kern


def kernel(x, edge_index, beta):
    n, d = x.shape
    e = edge_index.shape[1]
    src = edge_index[0]
    dst = edge_index[1]
    sc_a = _make_sc_edge_weights(n, e, d)
    sc_b = _make_sc_aggregate(n, e, d)
    h = x
    nh = _tc_normalize(x)
    for i in range(beta.shape[0]):
        beta16 = jnp.full((LN,), beta[i], jnp.float32)
        w, denom = sc_a(nh, src, dst, beta16)
        parts = sc_b(h, src, dst, w, denom[0], denom[1])
        h, nh = _tc_combine(parts)
    return h


# speedup vs baseline: 9.8597x; 1.1160x over previous
import functools

import jax
import jax.numpy as jnp
from jax import lax
from jax.experimental import pallas as pl
from jax.experimental.pallas import tpu as pltpu
from jax.experimental.pallas import tpu_sc as plsc

NC = 2
NS = 16
LN = 16
NW = NC * NS
CHUNK_A = 192
CHUNK_B = 80


def _norm_body(x_ref, nh_ref):
    x = x_ref[...]
    s = jnp.sum(x * x, axis=1, keepdims=True)
    nh_ref[...] = x / jnp.maximum(jnp.sqrt(s), 1e-12)


def _tc_normalize(x):
    n, d = x.shape
    blk = 1000
    return pl.pallas_call(
        _norm_body,
        grid=(n // blk,),
        in_specs=[pl.BlockSpec((blk, d), lambda i: (i, 0))],
        out_specs=pl.BlockSpec((blk, d), lambda i: (i, 0)),
        out_shape=jax.ShapeDtypeStruct((n, d), jnp.float32),
    )(x)


def _combine_body(parts_ref, h_ref, nh_ref):
    h = jnp.maximum(parts_ref[0] + parts_ref[1], 0.0)
    h_ref[...] = h
    s = jnp.sum(h * h, axis=1, keepdims=True)
    nh_ref[...] = h / jnp.maximum(jnp.sqrt(s), 1e-12)


def _tc_combine(parts):
    _, n, d = parts.shape
    blk = 1000
    return pl.pallas_call(
        _combine_body,
        grid=(n // blk,),
        in_specs=[pl.BlockSpec((2, blk, d), lambda i: (0, i, 0))],
        out_specs=(pl.BlockSpec((blk, d), lambda i: (i, 0)),
                   pl.BlockSpec((blk, d), lambda i: (i, 0))),
        out_shape=(jax.ShapeDtypeStruct((n, d), jnp.float32),
                   jax.ShapeDtypeStruct((n, d), jnp.float32)),
    )(parts)


def _sc_mesh():
    return plsc.VectorSubcoreMesh(
        core_axis_name="c", subcore_axis_name="s",
        num_cores=NC, num_subcores=NS)


@functools.lru_cache(maxsize=None)
def _make_sc_edge_weights(n, e, d):
    epw = e // NW
    nchunk = epw // CHUNK_A
    tail = epw - nchunk * CHUNK_A
    assert tail == LN
    kd = d // LN

    @functools.partial(
        pl.kernel,
        out_type=(jax.ShapeDtypeStruct((e,), jnp.float32),
                  jax.ShapeDtypeStruct((NC, n), jnp.float32)),
        mesh=_sc_mesh(),
        compiler_params=pltpu.CompilerParams(needs_layout_passes=False),
        scratch_types=[
            pltpu.VMEM((CHUNK_A,), jnp.int32),
            pltpu.VMEM((CHUNK_A,), jnp.int32),
            pltpu.VMEM((CHUNK_A,), jnp.int32),
            pltpu.VMEM((CHUNK_A,), jnp.int32),
            pltpu.VMEM((CHUNK_A, d), jnp.float32),
            pltpu.VMEM((CHUNK_A, d), jnp.float32),
            pltpu.VMEM((CHUNK_A, d), jnp.float32),
            pltpu.VMEM((CHUNK_A, d), jnp.float32),
            pltpu.VMEM((CHUNK_A,), jnp.float32),
            pltpu.VMEM((CHUNK_A,), jnp.float32),
            pltpu.VMEM((LN * LN,), jnp.float32),
            pltpu.VMEM((LN,), jnp.float32),
            pltpu.VMEM((n,), jnp.float32),
            pltpu.VMEM((2, LN), jnp.int32),
            pltpu.VMEM_SHARED((n,), jnp.float32),
            pltpu.SemaphoreType.DMA,
            pltpu.SemaphoreType.DMA,
            pltpu.SemaphoreType.DMA,
            pltpu.SemaphoreType.DMA,
            pltpu.SemaphoreType.DMA,
            pltpu.SemaphoreType.DMA,
            pltpu.SemaphoreType.DMA,
            pltpu.SemaphoreType.DMA,
        ],
    )
    def kern(nh_hbm, src_hbm, dst_hbm, beta_hbm, w_hbm, denom_hbm,
             sv0, sv1, dv0, dv1, sr0, sr1, dr0, dr1, w0, w1, accv, betav,
             nbuf, tiv, denom_sh, gs0, gs1, gd0, gd1, ss0, ss1, ws0, ws1):
        cid = lax.axis_index("c")
        sid = lax.axis_index("s")
        wid = cid * NS + sid
        svs = (sv0, sv1)
        dvs = (dv0, dv1)
        srowss = (sr0, sr1)
        drowss = (dr0, dr1)
        wvs = (w0, w1)
        gssems = (gs0, gs1)
        gdsems = (gd0, gd1)
        ssems = (ss0, ss1)
        wsems = (ws0, ws1)

        @pl.when(sid == 0)
        def _zero_denom():
            def zb(i, carry):
                nbuf[pl.ds(i * LN, LN)] = jnp.zeros((LN,), jnp.float32)
                return carry
            lax.fori_loop(0, n // LN, zb, 0)
            pltpu.sync_copy(nbuf, denom_sh)

        pltpu.sync_copy(beta_hbm, betav)
        plsc.subcore_barrier()

        bvec = betav[...]
        absb = jnp.abs(bvec)
        base0 = wid * epw
        lane = lax.broadcasted_iota(jnp.int32, (LN,), 0)

        def load_idx(b, j):
            base = base0 + j * CHUNK_A
            pltpu.sync_copy(src_hbm.at[pl.ds(base, CHUNK_A)], svs[b])
            pltpu.sync_copy(dst_hbm.at[pl.ds(base, CHUNK_A)], dvs[b])

        def issue_gathers(b):
            pltpu.async_copy(nh_hbm.at[svs[b]], srowss[b], gssems[b])
            pltpu.async_copy(nh_hbm.at[dvs[b]], drowss[b], gdsems[b])

        def wait_gathers(b):
            pltpu.make_async_copy(
                nh_hbm.at[svs[b]], srowss[b], gssems[b]).wait()
            pltpu.make_async_copy(
                nh_hbm.at[dvs[b]], drowss[b], gdsems[b]).wait()

        def issue_scatter(b):
            pltpu.async_copy(
                wvs[b], denom_sh.at[dvs[b]], ssems[b], add=True)

        def wait_scatter(b):
            pltpu.make_async_copy(
                wvs[b], denom_sh.at[dvs[b]], ssems[b]).wait()

        def issue_wstore(b, j):
            base = base0 + j * CHUNK_A
            pltpu.async_copy(wvs[b], w_hbm.at[pl.ds(base, CHUNK_A)], wsems[b])

        def wait_wstore(b, j):
            base = base0 + j * CHUNK_A
            pltpu.make_async_copy(
                wvs[b], w_hbm.at[pl.ds(base, CHUNK_A)], wsems[b]).wait()

        def compute(srows, drows, wv_b, ngroups):
            def grp(g, c2):
                for i in range(LN):
                    ei = g * LN + i
                    acc = srows[ei, pl.ds(0, LN)] * drows[ei, pl.ds(0, LN)]
                    for k in range(1, kd):
                        acc = acc + (srows[ei, pl.ds(k * LN, LN)]
                                     * drows[ei, pl.ds(k * LN, LN)])
                    accv[pl.ds(i * LN, LN)] = acc
                ii = lane * LN
                tot = plsc.load_gather(accv, [ii])
                for l in range(1, LN):
                    tot = tot + plsc.load_gather(accv, [ii + l])
                wv_b[pl.ds(g * LN, LN)] = jnp.exp(bvec * tot - absb)
                return c2
            lax.fori_loop(0, ngroups, grp, 0)

        load_idx(0, 0)
        issue_gathers(0)

        def pair(j2, carry):
            for b in (0, 1):
                j = 2 * j2 + b
                wait_gathers(b)

                @pl.when(j >= 1)
                def _():
                    wait_scatter(b ^ 1)

                @pl.when(j + 1 < nchunk)
                def _():
                    load_idx(b ^ 1, j + 1)
                    issue_gathers(b ^ 1)

                @pl.when(j >= 2)
                def _():
                    wait_wstore(b, j - 2)
                compute(srowss[b], drowss[b], wvs[b], CHUNK_A // LN)
                issue_wstore(b, j)
                issue_scatter(b)
            return carry
        lax.fori_loop(0, nchunk // 2, pair, 0)

        wait_scatter(1)
        wait_wstore(0, nchunk - 2)
        wait_wstore(1, nchunk - 1)
        tbase = base0 + nchunk * CHUNK_A
        pltpu.sync_copy(src_hbm.at[pl.ds(tbase, tail)], tiv.at[0])
        pltpu.sync_copy(dst_hbm.at[pl.ds(tbase, tail)], tiv.at[1])
        tsrc = tiv.at[0]
        tdst = tiv.at[1]
        pltpu.async_copy(
            nh_hbm.at[tsrc], sr0.at[pl.ds(0, tail)], gs0).wait()
        pltpu.async_copy(
            nh_hbm.at[tdst], dr0.at[pl.ds(0, tail)], gd0).wait()
        compute(sr0, dr0, w0, 1)
        pltpu.sync_copy(w0.at[pl.ds(0, tail)], w_hbm.at[pl.ds(tbase, tail)])
        pltpu.sync_copy(w0.at[pl.ds(0, tail)], denom_sh.at[tdst], add=True)

        plsc.subcore_barrier()

        @pl.when(sid == 0)
        def _dump():
            pltpu.sync_copy(denom_sh, nbuf)
            pltpu.sync_copy(nbuf, denom_hbm.at[cid])

    return kern


@functools.lru_cache(maxsize=None)
def _make_sc_aggregate(n, e, d):
    epw = e // NW
    nchunk = epw // CHUNK_B
    kd = d // LN
    nzc = n // CHUNK_B

    @functools.partial(
        pl.kernel,
        out_type=jax.ShapeDtypeStruct((NC, n, d), jnp.float32),
        mesh=_sc_mesh(),
        compiler_params=pltpu.CompilerParams(needs_layout_passes=False),
        scratch_types=[
            pltpu.VMEM((2, CHUNK_B), jnp.int32),
            pltpu.VMEM((2, CHUNK_B), jnp.int32),
            pltpu.VMEM((CHUNK_B,), jnp.float32),
            pltpu.VMEM((CHUNK_B,), jnp.float32),
            pltpu.VMEM((CHUNK_B, d), jnp.float32),
            pltpu.VMEM((CHUNK_B, d), jnp.float32),
            pltpu.VMEM((n,), jnp.float32),
            pltpu.VMEM((n,), jnp.float32),
            pltpu.VMEM_SHARED((n, d), jnp.float32),
            pltpu.SemaphoreType.DMA,
            pltpu.SemaphoreType.DMA,
            pltpu.SemaphoreType.DMA,
            pltpu.SemaphoreType.DMA,
        ],
    )
    def kern(h_hbm, src_hbm, dst_hbm, w_hbm, d0_hbm, d1_hbm, out_hbm,
             ei0, ei1, w0, w1, rows0, rows1, d0buf, d1buf, out_sh,
             g0, g1, s0, s1):
        cid = lax.axis_index("c")
        sid = lax.axis_index("s")
        wid = cid * NS + sid
        eis = (ei0, ei1)
        wvs = (w0, w1)
        rowss = (rows0, rows1)
        gsems = (g0, g1)
        ssems = (s0, s1)

        def zr(i, carry):
            rows0[i // kd, pl.ds((i % kd) * LN, LN)] = jnp.zeros(
                (LN,), jnp.float32)
            return carry
        lax.fori_loop(0, CHUNK_B * kd, zr, 0)
        for t in range((nzc + NS - 1) // NS):
            k = sid + t * NS

            @pl.when(k < nzc)
            def _zc():
                pltpu.sync_copy(rows0, out_sh.at[pl.ds(k * CHUNK_B, CHUNK_B)])
        pltpu.sync_copy(d0_hbm, d0buf)
        pltpu.sync_copy(d1_hbm, d1buf)
        plsc.subcore_barrier()

        base0 = wid * epw

        def load_idx(b, j):
            base = base0 + j * CHUNK_B
            pltpu.sync_copy(src_hbm.at[pl.ds(base, CHUNK_B)], eis[b].at[0])
            pltpu.sync_copy(dst_hbm.at[pl.ds(base, CHUNK_B)], eis[b].at[1])
            pltpu.sync_copy(w_hbm.at[pl.ds(base, CHUNK_B)], wvs[b])

        def issue_gather(b):
            pltpu.async_copy(h_hbm.at[eis[b].at[0]], rowss[b], gsems[b])

        def wait_gather(b):
            pltpu.make_async_copy(
                h_hbm.at[eis[b].at[0]], rowss[b], gsems[b]).wait()

        def issue_scatter(b):
            pltpu.async_copy(
                rowss[b], out_sh.at[eis[b].at[1]], ssems[b], add=True)

        def wait_scatter(b):
            pltpu.make_async_copy(
                rowss[b], out_sh.at[eis[b].at[1]], ssems[b]).wait()

        def compute(b):
            ei_b, wv_b, rows_b = eis[b], wvs[b], rowss[b]

            def pv(g, c2):
                sl = pl.ds(g * LN, LN)
                dv = ei_b[1, sl]
                a0 = plsc.load_gather(d0buf, [dv])
                a1 = plsc.load_gather(d1buf, [dv])
                wv_b[sl] = wv_b[sl] / (a0 + a1)
                return c2
            lax.fori_loop(0, CHUNK_B // LN, pv, 0)

            def scale(eidx, c2):
                p = plsc.load_gather(
                    wv_b, [jnp.full((LN,), eidx, jnp.int32)])
                for k in range(kd):
                    sl = pl.ds(k * LN, LN)
                    rows_b[eidx, sl] = rows_b[eidx, sl] * p
                return c2
            lax.fori_loop(0, CHUNK_B, scale, 0)

        load_idx(0, 0)
        issue_gather(0)

        def pair(j2, carry):
            for b in (0, 1):
                j = 2 * j2 + b
                wait_gather(b)

                @pl.when(j >= 1)
                def _():
                    wait_scatter(b ^ 1)
                load_idx(b ^ 1, j + 1)
                issue_gather(b ^ 1)
                compute(b)
                issue_scatter(b)
            return carry
        lax.fori_loop(0, (nchunk - 1) // 2, pair, 0)

        wait_gather(0)
        wait_scatter(1)
        compute(0)
        pltpu.sync_copy(rows0, out_sh.at[ei0.at[1]], add=True)

        plsc.subcore_barrier()
        for t in range((nzc + NS - 1) // NS):
            k = sid + t * NS

            @pl.when(k < nzc)
            def _dump():
                pltpu.sync_copy(out_sh.at[pl.ds(k * CHUNK_B, CHUNK_B)], rows0)
                pltpu.sync_copy(rows0, out_hbm.at[cid, pl.ds(k * CHUNK_B, CHUNK_B)])

    return kern


def kernel(x, edge_index, beta):
    n, d = x.shape
    e = edge_index.shape[1]
    src = edge_index[0]
    dst = edge_index[1]
    sc_a = _make_sc_edge_weights(n, e, d)
    sc_b = _make_sc_aggregate(n, e, d)
    h = x
    nh = _tc_normalize(x)
    for i in range(beta.shape[0]):
        beta16 = jnp.full((LN,), beta[i], jnp.float32)
        w, denom = sc_a(nh, src, dst, beta16)
        parts = sc_b(h, src, dst, w, denom[0], denom[1])
        h, nh = _tc_combine(parts)
    return h


# speedup vs baseline: 11.0175x; 1.1174x over previous
import functools

import jax
import jax.numpy as jnp
from jax import lax
from jax.experimental import pallas as pl
from jax.experimental.pallas import tpu as pltpu
from jax.experimental.pallas import tpu_sc as plsc

NC = 2
NS = 16
LN = 16
NW = NC * NS
CHUNK_A = 208
CHUNK_B = 144
TAIL_B = 64
NPAD = 10240


def _norm_body(x_ref, nh_ref):
    x = x_ref[...]
    s = jnp.sum(x * x, axis=1, keepdims=True)
    nh_ref[...] = x / jnp.maximum(jnp.sqrt(s), 1e-12)


def _tc_normalize(x):
    n, d = x.shape
    blk = 1000
    return pl.pallas_call(
        _norm_body,
        grid=(n // blk,),
        in_specs=[pl.BlockSpec((blk, d), lambda i: (i, 0))],
        out_specs=pl.BlockSpec((blk, d), lambda i: (i, 0)),
        out_shape=jax.ShapeDtypeStruct((n, d), jnp.float32),
    )(x)


def _combine_body(parts_ref, h_ref, nh_ref):
    h = jnp.maximum(parts_ref[0] + parts_ref[1], 0.0)
    h_ref[...] = h
    s = jnp.sum(h * h, axis=1, keepdims=True)
    nh_ref[...] = h / jnp.maximum(jnp.sqrt(s), 1e-12)


def _tc_combine(parts):
    _, n, d = parts.shape
    blk = 1000
    return pl.pallas_call(
        _combine_body,
        grid=(n // blk,),
        in_specs=[pl.BlockSpec((2, blk, d), lambda i: (0, i, 0))],
        out_specs=(pl.BlockSpec((blk, d), lambda i: (i, 0)),
                   pl.BlockSpec((blk, d), lambda i: (i, 0))),
        out_shape=(jax.ShapeDtypeStruct((n, d), jnp.float32),
                   jax.ShapeDtypeStruct((n, d), jnp.float32)),
    )(parts)


def _dinv_body(dn_ref, o_ref):
    o_ref[...] = 1.0 / (dn_ref[0] + dn_ref[1])


def _tc_dinv(parts):
    _, r, c = parts.shape
    blk = 8
    return pl.pallas_call(
        _dinv_body,
        grid=(r // blk,),
        in_specs=[pl.BlockSpec((2, blk, c), lambda i: (0, i, 0))],
        out_specs=pl.BlockSpec((blk, c), lambda i: (i, 0)),
        out_shape=jax.ShapeDtypeStruct((r, c), jnp.float32),
    )(parts)


def _sc_mesh():
    return plsc.VectorSubcoreMesh(
        core_axis_name="c", subcore_axis_name="s",
        num_cores=NC, num_subcores=NS)


@functools.lru_cache(maxsize=None)
def _make_sc_edge_weights(n, e, d):
    epw = e // NW
    nchunk = epw // CHUNK_A
    tail = epw - nchunk * CHUNK_A
    assert tail == LN
    kd = d // LN

    @functools.partial(
        pl.kernel,
        out_type=(jax.ShapeDtypeStruct((e,), jnp.float32),
                  jax.ShapeDtypeStruct((NC, NPAD), jnp.float32)),
        mesh=_sc_mesh(),
        compiler_params=pltpu.CompilerParams(needs_layout_passes=False),
        scratch_types=[
            pltpu.VMEM((CHUNK_A,), jnp.int32),
            pltpu.VMEM((CHUNK_A,), jnp.int32),
            pltpu.VMEM((CHUNK_A,), jnp.int32),
            pltpu.VMEM((CHUNK_A,), jnp.int32),
            pltpu.VMEM((CHUNK_A, d), jnp.float32),
            pltpu.VMEM((CHUNK_A, d), jnp.float32),
            pltpu.VMEM((CHUNK_A, d), jnp.float32),
            pltpu.VMEM((CHUNK_A, d), jnp.float32),
            pltpu.VMEM((CHUNK_A,), jnp.float32),
            pltpu.VMEM((CHUNK_A,), jnp.float32),
            pltpu.VMEM((LN * LN,), jnp.float32),
            pltpu.VMEM((LN,), jnp.float32),
            pltpu.VMEM((NPAD,), jnp.float32),
            pltpu.VMEM((2, LN), jnp.int32),
            pltpu.VMEM_SHARED((NPAD,), jnp.float32),
            pltpu.SemaphoreType.DMA,
            pltpu.SemaphoreType.DMA,
            pltpu.SemaphoreType.DMA,
            pltpu.SemaphoreType.DMA,
            pltpu.SemaphoreType.DMA,
            pltpu.SemaphoreType.DMA,
            pltpu.SemaphoreType.DMA,
            pltpu.SemaphoreType.DMA,
        ],
    )
    def kern(nh_hbm, src_hbm, dst_hbm, beta_hbm, w_hbm, denom_hbm,
             sv0, sv1, dv0, dv1, sr0, sr1, dr0, dr1, w0, w1, accv, betav,
             nbuf, tiv, denom_sh, gs0, gs1, gd0, gd1, ss0, ss1, ws0, ws1):
        cid = lax.axis_index("c")
        sid = lax.axis_index("s")
        wid = cid * NS + sid
        svs = (sv0, sv1)
        dvs = (dv0, dv1)
        srowss = (sr0, sr1)
        drowss = (dr0, dr1)
        wvs = (w0, w1)
        gssems = (gs0, gs1)
        gdsems = (gd0, gd1)
        ssems = (ss0, ss1)
        wsems = (ws0, ws1)

        @pl.when(sid == 0)
        def _zero_denom():
            def zb(i, carry):
                nbuf[pl.ds(i * LN, LN)] = jnp.zeros((LN,), jnp.float32)
                return carry
            lax.fori_loop(0, NPAD // LN, zb, 0)
            pltpu.sync_copy(nbuf, denom_sh)

        pltpu.sync_copy(beta_hbm, betav)
        plsc.subcore_barrier()

        bvec = betav[...]
        absb = jnp.abs(bvec)
        base0 = wid * epw
        lane = lax.broadcasted_iota(jnp.int32, (LN,), 0)

        def load_idx(b, j):
            base = base0 + j * CHUNK_A
            pltpu.sync_copy(src_hbm.at[pl.ds(base, CHUNK_A)], svs[b])
            pltpu.sync_copy(dst_hbm.at[pl.ds(base, CHUNK_A)], dvs[b])

        def issue_gathers(b):
            pltpu.async_copy(nh_hbm.at[svs[b]], srowss[b], gssems[b])
            pltpu.async_copy(nh_hbm.at[dvs[b]], drowss[b], gdsems[b])

        def wait_gathers(b):
            pltpu.make_async_copy(
                nh_hbm.at[svs[b]], srowss[b], gssems[b]).wait()
            pltpu.make_async_copy(
                nh_hbm.at[dvs[b]], drowss[b], gdsems[b]).wait()

        def issue_scatter(b):
            pltpu.async_copy(
                wvs[b], denom_sh.at[dvs[b]], ssems[b], add=True)

        def wait_scatter(b):
            pltpu.make_async_copy(
                wvs[b], denom_sh.at[dvs[b]], ssems[b]).wait()

        def issue_wstore(b, j):
            base = base0 + j * CHUNK_A
            pltpu.async_copy(wvs[b], w_hbm.at[pl.ds(base, CHUNK_A)], wsems[b])

        def wait_wstore(b, j):
            base = base0 + j * CHUNK_A
            pltpu.make_async_copy(
                wvs[b], w_hbm.at[pl.ds(base, CHUNK_A)], wsems[b]).wait()

        def compute(srows, drows, wv_b, ngroups):
            def grp(g, c2):
                for i in range(LN):
                    ei = g * LN + i
                    acc = srows[ei, pl.ds(0, LN)] * drows[ei, pl.ds(0, LN)]
                    for k in range(1, kd):
                        acc = acc + (srows[ei, pl.ds(k * LN, LN)]
                                     * drows[ei, pl.ds(k * LN, LN)])
                    accv[pl.ds(i * LN, LN)] = acc
                ii = lane * LN
                tot = plsc.load_gather(accv, [ii])
                for l in range(1, LN):
                    tot = tot + plsc.load_gather(accv, [ii + l])
                wv_b[pl.ds(g * LN, LN)] = jnp.exp(bvec * tot - absb)
                return c2
            lax.fori_loop(0, ngroups, grp, 0)

        load_idx(0, 0)
        issue_gathers(0)

        def pair(j2, carry):
            for b in (0, 1):
                j = 2 * j2 + b
                wait_gathers(b)

                @pl.when(j >= 1)
                def _():
                    wait_scatter(b ^ 1)

                @pl.when(j + 1 < nchunk)
                def _():
                    load_idx(b ^ 1, j + 1)
                    issue_gathers(b ^ 1)

                @pl.when(j >= 2)
                def _():
                    wait_wstore(b, j - 2)
                compute(srowss[b], drowss[b], wvs[b], CHUNK_A // LN)
                issue_wstore(b, j)
                issue_scatter(b)
            return carry
        lax.fori_loop(0, nchunk // 2, pair, 0)

        wait_scatter(1)
        wait_wstore(0, nchunk - 2)
        wait_wstore(1, nchunk - 1)
        tbase = base0 + nchunk * CHUNK_A
        pltpu.sync_copy(src_hbm.at[pl.ds(tbase, tail)], tiv.at[0])
        pltpu.sync_copy(dst_hbm.at[pl.ds(tbase, tail)], tiv.at[1])
        tsrc = tiv.at[0]
        tdst = tiv.at[1]
        pltpu.async_copy(
            nh_hbm.at[tsrc], sr0.at[pl.ds(0, tail)], gs0).wait()
        pltpu.async_copy(
            nh_hbm.at[tdst], dr0.at[pl.ds(0, tail)], gd0).wait()
        compute(sr0, dr0, w0, 1)
        pltpu.sync_copy(w0.at[pl.ds(0, tail)], w_hbm.at[pl.ds(tbase, tail)])
        pltpu.sync_copy(w0.at[pl.ds(0, tail)], denom_sh.at[tdst], add=True)

        plsc.subcore_barrier()

        @pl.when(sid == 0)
        def _dump():
            pltpu.sync_copy(denom_sh, nbuf)
            pltpu.sync_copy(nbuf, denom_hbm.at[cid])

    return kern


@functools.lru_cache(maxsize=None)
def _make_sc_aggregate(n, e, d):
    epw = e // NW
    nchunk = epw // CHUNK_B
    assert epw - nchunk * CHUNK_B == TAIL_B and TAIL_B % LN == 0
    kd = d // LN
    nzc = n // CHUNK_B
    zrem = n - nzc * CHUNK_B
    assert zrem % 8 == 0

    @functools.partial(
        pl.kernel,
        out_type=jax.ShapeDtypeStruct((NC, n, d), jnp.float32),
        mesh=_sc_mesh(),
        compiler_params=pltpu.CompilerParams(needs_layout_passes=False),
        scratch_types=[
            pltpu.VMEM((CHUNK_B,), jnp.int32),
            pltpu.VMEM((CHUNK_B,), jnp.int32),
            pltpu.VMEM((CHUNK_B,), jnp.int32),
            pltpu.VMEM((CHUNK_B,), jnp.int32),
            pltpu.VMEM((CHUNK_B,), jnp.float32),
            pltpu.VMEM((CHUNK_B,), jnp.float32),
            pltpu.VMEM((CHUNK_B, d), jnp.float32),
            pltpu.VMEM((CHUNK_B, d), jnp.float32),
            pltpu.VMEM((n,), jnp.float32),
            pltpu.VMEM((TAIL_B,), jnp.int32),
            pltpu.VMEM((TAIL_B,), jnp.int32),
            pltpu.VMEM_SHARED((n, d), jnp.float32),
            pltpu.SemaphoreType.DMA,
            pltpu.SemaphoreType.DMA,
            pltpu.SemaphoreType.DMA,
            pltpu.SemaphoreType.DMA,
        ],
    )
    def kern(h_hbm, src_hbm, dst_hbm, w_hbm, dinv_hbm, out_hbm,
             sv0, sv1, dv0, dv1, w0, w1, rows0, rows1, dinvbuf, tsv, tdv,
             out_sh, g0, g1, s0, s1):
        cid = lax.axis_index("c")
        sid = lax.axis_index("s")
        wid = cid * NS + sid
        svs = (sv0, sv1)
        dvs = (dv0, dv1)
        wvs = (w0, w1)
        rowss = (rows0, rows1)
        gsems = (g0, g1)
        ssems = (s0, s1)

        def zr(i, carry):
            rows0[i // kd, pl.ds((i % kd) * LN, LN)] = jnp.zeros(
                (LN,), jnp.float32)
            return carry
        lax.fori_loop(0, CHUNK_B * kd, zr, 0)
        for t in range((nzc + NS - 1) // NS):
            k = sid + t * NS

            @pl.when(k < nzc)
            def _zc():
                pltpu.sync_copy(rows0, out_sh.at[pl.ds(k * CHUNK_B, CHUNK_B)])

        @pl.when(sid == 0)
        def _zrem():
            pltpu.sync_copy(rows0.at[pl.ds(0, zrem)],
                            out_sh.at[pl.ds(nzc * CHUNK_B, zrem)])
        pltpu.sync_copy(dinv_hbm.at[pl.ds(0, n)], dinvbuf)
        plsc.subcore_barrier()

        base0 = wid * epw

        def load_idx(b, j):
            base = base0 + j * CHUNK_B
            pltpu.sync_copy(src_hbm.at[pl.ds(base, CHUNK_B)], svs[b])
            pltpu.sync_copy(dst_hbm.at[pl.ds(base, CHUNK_B)], dvs[b])
            pltpu.sync_copy(w_hbm.at[pl.ds(base, CHUNK_B)], wvs[b])

        def issue_gather(b):
            pltpu.async_copy(h_hbm.at[svs[b]], rowss[b], gsems[b])

        def wait_gather(b):
            pltpu.make_async_copy(h_hbm.at[svs[b]], rowss[b], gsems[b]).wait()

        def issue_scatter(b):
            pltpu.async_copy(
                rowss[b], out_sh.at[dvs[b]], ssems[b], add=True)

        def wait_scatter(b):
            pltpu.make_async_copy(
                rowss[b], out_sh.at[dvs[b]], ssems[b]).wait()

        def compute(dstv, wv_b, rows_b, ngroups):
            def pv(g, c2):
                sl = pl.ds(g * LN, LN)
                dv16 = dstv[sl]
                wv_b[sl] = wv_b[sl] * plsc.load_gather(dinvbuf, [dv16])
                return c2
            lax.fori_loop(0, ngroups, pv, 0)

            def scale(eidx, c2):
                p = plsc.load_gather(
                    wv_b, [jnp.full((LN,), eidx, jnp.int32)])
                for k in range(kd):
                    sl = pl.ds(k * LN, LN)
                    rows_b[eidx, sl] = rows_b[eidx, sl] * p
                return c2
            lax.fori_loop(0, ngroups * LN, scale, 0)

        load_idx(0, 0)
        issue_gather(0)

        def pair(j2, carry):
            for b in (0, 1):
                j = 2 * j2 + b
                wait_gather(b)

                @pl.when(j >= 1)
                def _():
                    wait_scatter(b ^ 1)

                @pl.when(j + 1 < nchunk)
                def _():
                    load_idx(b ^ 1, j + 1)
                    issue_gather(b ^ 1)
                compute(dvs[b], wvs[b], rowss[b], CHUNK_B // LN)
                issue_scatter(b)
            return carry
        lax.fori_loop(0, (nchunk - 1) // 2, pair, 0)

        wait_gather(0)
        wait_scatter(1)
        tbase = base0 + nchunk * CHUNK_B
        pltpu.sync_copy(src_hbm.at[pl.ds(tbase, TAIL_B)], tsv)
        pltpu.sync_copy(dst_hbm.at[pl.ds(tbase, TAIL_B)], tdv)
        pltpu.sync_copy(w_hbm.at[pl.ds(tbase, TAIL_B)],
                        w1.at[pl.ds(0, TAIL_B)])
        pltpu.async_copy(h_hbm.at[tsv], rows1.at[pl.ds(0, TAIL_B)], g1)
        compute(dv0, w0, rows0, CHUNK_B // LN)
        issue_scatter(0)
        pltpu.make_async_copy(
            h_hbm.at[tsv], rows1.at[pl.ds(0, TAIL_B)], g1).wait()
        compute(tdv, w1, rows1, TAIL_B // LN)
        wait_scatter(0)
        pltpu.sync_copy(rows1.at[pl.ds(0, TAIL_B)], out_sh.at[tdv], add=True)

        plsc.subcore_barrier()
        for t in range((nzc + NS - 1) // NS):
            k = sid + t * NS

            @pl.when(k < nzc)
            def _dump():
                pltpu.sync_copy(out_sh.at[pl.ds(k * CHUNK_B, CHUNK_B)], rows0)
                pltpu.sync_copy(
                    rows0, out_hbm.at[cid, pl.ds(k * CHUNK_B, CHUNK_B)])

        @pl.when(sid == 0)
        def _drem():
            pltpu.sync_copy(out_sh.at[pl.ds(nzc * CHUNK_B, zrem)],
                            rows0.at[pl.ds(0, zrem)])
            pltpu.sync_copy(rows0.at[pl.ds(0, zrem)],
                            out_hbm.at[cid, pl.ds(nzc * CHUNK_B, zrem)])

    return kern


def kernel(x, edge_index, beta):
    n, d = x.shape
    e = edge_index.shape[1]
    src = edge_index[0]
    dst = edge_index[1]
    sc_a = _make_sc_edge_weights(n, e, d)
    sc_b = _make_sc_aggregate(n, e, d)
    h = x
    nh = _tc_normalize(x)
    for i in range(beta.shape[0]):
        beta16 = jnp.full((LN,), beta[i], jnp.float32)
        w, denom = sc_a(nh, src, dst, beta16)
        dinv = _tc_dinv(denom.reshape(NC, NPAD // 128, 128)).reshape(NPAD)
        parts = sc_b(h, src, dst, w, dinv)
        h, nh = _tc_combine(parts)
    return h


# speedup vs baseline: 11.1060x; 1.0080x over previous
import functools

import jax
import jax.numpy as jnp
from jax import lax
from jax.experimental import pallas as pl
from jax.experimental.pallas import tpu as pltpu
from jax.experimental.pallas import tpu_sc as plsc

NC = 2
NS = 16
LN = 16
NW = NC * NS
CHUNK_A = 208
CHUNK_B = 144
TAIL_B = 64
NPAD = 10240


def _norm_body(x_ref, nh_ref):
    x = x_ref[...]
    s = jnp.sum(x * x, axis=1, keepdims=True)
    nh_ref[...] = x / jnp.maximum(jnp.sqrt(s), 1e-12)


def _tc_normalize(x):
    n, d = x.shape
    blk = 1000
    return pl.pallas_call(
        _norm_body,
        grid=(n // blk,),
        in_specs=[pl.BlockSpec((blk, d), lambda i: (i, 0))],
        out_specs=pl.BlockSpec((blk, d), lambda i: (i, 0)),
        out_shape=jax.ShapeDtypeStruct((n, d), jnp.float32),
    )(x)


def _combine_body(parts_ref, h_ref, nh_ref):
    h = jnp.maximum(parts_ref[0] + parts_ref[1], 0.0)
    h_ref[...] = h
    s = jnp.sum(h * h, axis=1, keepdims=True)
    nh_ref[...] = h / jnp.maximum(jnp.sqrt(s), 1e-12)


def _tc_combine(parts):
    _, n, d = parts.shape
    blk = 1000
    return pl.pallas_call(
        _combine_body,
        grid=(n // blk,),
        in_specs=[pl.BlockSpec((2, blk, d), lambda i: (0, i, 0))],
        out_specs=(pl.BlockSpec((blk, d), lambda i: (i, 0)),
                   pl.BlockSpec((blk, d), lambda i: (i, 0))),
        out_shape=(jax.ShapeDtypeStruct((n, d), jnp.float32),
                   jax.ShapeDtypeStruct((n, d), jnp.float32)),
    )(parts)


def _dinv_body(dn_ref, o_ref):
    o_ref[...] = 1.0 / (dn_ref[0] + dn_ref[1])


def _tc_dinv(parts):
    _, r, c = parts.shape
    blk = 8
    return pl.pallas_call(
        _dinv_body,
        grid=(r // blk,),
        in_specs=[pl.BlockSpec((2, blk, c), lambda i: (0, i, 0))],
        out_specs=pl.BlockSpec((blk, c), lambda i: (i, 0)),
        out_shape=jax.ShapeDtypeStruct((r, c), jnp.float32),
    )(parts)


def _sc_mesh():
    return plsc.VectorSubcoreMesh(
        core_axis_name="c", subcore_axis_name="s",
        num_cores=NC, num_subcores=NS)


@functools.lru_cache(maxsize=None)
def _make_sc_edge_weights(n, e, d):
    epw = e // NW
    nchunk = epw // CHUNK_A
    tail = epw - nchunk * CHUNK_A
    assert tail == LN
    kd = d // LN

    @functools.partial(
        pl.kernel,
        out_type=(jax.ShapeDtypeStruct((e,), jnp.float32),
                  jax.ShapeDtypeStruct((NC, NPAD), jnp.float32)),
        mesh=_sc_mesh(),
        compiler_params=pltpu.CompilerParams(needs_layout_passes=False),
        scratch_types=[
            pltpu.VMEM((CHUNK_A,), jnp.int32),
            pltpu.VMEM((CHUNK_A,), jnp.int32),
            pltpu.VMEM((CHUNK_A,), jnp.int32),
            pltpu.VMEM((CHUNK_A,), jnp.int32),
            pltpu.VMEM((CHUNK_A, d), jnp.float32),
            pltpu.VMEM((CHUNK_A, d), jnp.float32),
            pltpu.VMEM((CHUNK_A, d), jnp.float32),
            pltpu.VMEM((CHUNK_A, d), jnp.float32),
            pltpu.VMEM((CHUNK_A,), jnp.float32),
            pltpu.VMEM((CHUNK_A,), jnp.float32),
            pltpu.VMEM((LN * LN,), jnp.float32),
            pltpu.VMEM((LN,), jnp.float32),
            pltpu.VMEM((NPAD,), jnp.float32),
            pltpu.VMEM((2, LN), jnp.int32),
            pltpu.VMEM_SHARED((NPAD,), jnp.float32),
            pltpu.SemaphoreType.DMA,
            pltpu.SemaphoreType.DMA,
            pltpu.SemaphoreType.DMA,
            pltpu.SemaphoreType.DMA,
            pltpu.SemaphoreType.DMA,
            pltpu.SemaphoreType.DMA,
            pltpu.SemaphoreType.DMA,
            pltpu.SemaphoreType.DMA,
        ],
    )
    def kern(nh_hbm, src_hbm, dst_hbm, beta_hbm, w_hbm, denom_hbm,
             sv0, sv1, dv0, dv1, sr0, sr1, dr0, dr1, w0, w1, accv, betav,
             nbuf, tiv, denom_sh, gs0, gs1, gd0, gd1, ss0, ss1, ws0, ws1):
        cid = lax.axis_index("c")
        sid = lax.axis_index("s")
        wid = cid * NS + sid
        svs = (sv0, sv1)
        dvs = (dv0, dv1)
        srowss = (sr0, sr1)
        drowss = (dr0, dr1)
        wvs = (w0, w1)
        gssems = (gs0, gs1)
        gdsems = (gd0, gd1)
        ssems = (ss0, ss1)
        wsems = (ws0, ws1)

        @pl.when(sid == 0)
        def _zero_denom():
            def zb(i, carry):
                nbuf[pl.ds(i * LN, LN)] = jnp.zeros((LN,), jnp.float32)
                return carry
            lax.fori_loop(0, NPAD // LN, zb, 0)
            pltpu.sync_copy(nbuf, denom_sh)

        pltpu.sync_copy(beta_hbm, betav)
        plsc.subcore_barrier()

        bvec = betav[...]
        absb = jnp.abs(bvec)
        base0 = wid * epw
        lane = lax.broadcasted_iota(jnp.int32, (LN,), 0)

        def load_idx(b, j):
            base = base0 + j * CHUNK_A
            pltpu.sync_copy(src_hbm.at[pl.ds(base, CHUNK_A)], svs[b])
            pltpu.sync_copy(dst_hbm.at[pl.ds(base, CHUNK_A)], dvs[b])

        def issue_gathers(b):
            pltpu.async_copy(nh_hbm.at[svs[b]], srowss[b], gssems[b])
            pltpu.async_copy(nh_hbm.at[dvs[b]], drowss[b], gdsems[b])

        def wait_gathers(b):
            pltpu.make_async_copy(
                nh_hbm.at[svs[b]], srowss[b], gssems[b]).wait()
            pltpu.make_async_copy(
                nh_hbm.at[dvs[b]], drowss[b], gdsems[b]).wait()

        def issue_scatter(b):
            pltpu.async_copy(
                wvs[b], denom_sh.at[dvs[b]], ssems[b], add=True)

        def wait_scatter(b):
            pltpu.make_async_copy(
                wvs[b], denom_sh.at[dvs[b]], ssems[b]).wait()

        def issue_wstore(b, j):
            base = base0 + j * CHUNK_A
            pltpu.async_copy(wvs[b], w_hbm.at[pl.ds(base, CHUNK_A)], wsems[b])

        def wait_wstore(b, j):
            base = base0 + j * CHUNK_A
            pltpu.make_async_copy(
                wvs[b], w_hbm.at[pl.ds(base, CHUNK_A)], wsems[b]).wait()

        def compute(srows, drows, wv_b, ngroups):
            def grp(g, c2):
                for i in range(LN):
                    ei = g * LN + i
                    acc = srows[ei, pl.ds(0, LN)] * drows[ei, pl.ds(0, LN)]
                    for k in range(1, kd):
                        acc = acc + (srows[ei, pl.ds(k * LN, LN)]
                                     * drows[ei, pl.ds(k * LN, LN)])
                    accv[pl.ds(i * LN, LN)] = acc
                ii = lane * LN
                tot0 = plsc.load_gather(accv, [ii])
                tot1 = plsc.load_gather(accv, [ii + 1])
                for l in range(2, LN, 2):
                    tot0 = tot0 + plsc.load_gather(accv, [ii + l])
                    tot1 = tot1 + plsc.load_gather(accv, [ii + l + 1])
                tot = tot0 + tot1
                wv_b[pl.ds(g * LN, LN)] = jnp.exp(bvec * tot - absb)
                return c2
            lax.fori_loop(0, ngroups, grp, 0)

        load_idx(0, 0)
        issue_gathers(0)

        def pair(j2, carry):
            for b in (0, 1):
                j = 2 * j2 + b
                wait_gathers(b)

                @pl.when(j >= 1)
                def _():
                    wait_scatter(b ^ 1)

                @pl.when(j + 1 < nchunk)
                def _():
                    load_idx(b ^ 1, j + 1)
                    issue_gathers(b ^ 1)

                @pl.when(j >= 2)
                def _():
                    wait_wstore(b, j - 2)
                compute(srowss[b], drowss[b], wvs[b], CHUNK_A // LN)
                issue_wstore(b, j)
                issue_scatter(b)
            return carry
        lax.fori_loop(0, nchunk // 2, pair, 0)

        wait_scatter(1)
        wait_wstore(0, nchunk - 2)
        wait_wstore(1, nchunk - 1)
        tbase = base0 + nchunk * CHUNK_A
        pltpu.sync_copy(src_hbm.at[pl.ds(tbase, tail)], tiv.at[0])
        pltpu.sync_copy(dst_hbm.at[pl.ds(tbase, tail)], tiv.at[1])
        tsrc = tiv.at[0]
        tdst = tiv.at[1]
        pltpu.async_copy(
            nh_hbm.at[tsrc], sr0.at[pl.ds(0, tail)], gs0).wait()
        pltpu.async_copy(
            nh_hbm.at[tdst], dr0.at[pl.ds(0, tail)], gd0).wait()
        compute(sr0, dr0, w0, 1)
        pltpu.sync_copy(w0.at[pl.ds(0, tail)], w_hbm.at[pl.ds(tbase, tail)])
        pltpu.sync_copy(w0.at[pl.ds(0, tail)], denom_sh.at[tdst], add=True)

        plsc.subcore_barrier()

        @pl.when(sid == 0)
        def _dump():
            pltpu.sync_copy(denom_sh, nbuf)
            pltpu.sync_copy(nbuf, denom_hbm.at[cid])

    return kern


@functools.lru_cache(maxsize=None)
def _make_sc_aggregate(n, e, d):
    epw = e // NW
    nchunk = epw // CHUNK_B
    assert epw - nchunk * CHUNK_B == TAIL_B and TAIL_B % LN == 0
    kd = d // LN
    nzc = n // CHUNK_B
    zrem = n - nzc * CHUNK_B
    assert zrem % 8 == 0

    @functools.partial(
        pl.kernel,
        out_type=jax.ShapeDtypeStruct((NC, n, d), jnp.float32),
        mesh=_sc_mesh(),
        compiler_params=pltpu.CompilerParams(needs_layout_passes=False),
        scratch_types=[
            pltpu.VMEM((CHUNK_B,), jnp.int32),
            pltpu.VMEM((CHUNK_B,), jnp.int32),
            pltpu.VMEM((CHUNK_B,), jnp.int32),
            pltpu.VMEM((CHUNK_B,), jnp.int32),
            pltpu.VMEM((CHUNK_B,), jnp.float32),
            pltpu.VMEM((CHUNK_B,), jnp.float32),
            pltpu.VMEM((CHUNK_B, d), jnp.float32),
            pltpu.VMEM((CHUNK_B, d), jnp.float32),
            pltpu.VMEM((n,), jnp.float32),
            pltpu.VMEM((TAIL_B,), jnp.int32),
            pltpu.VMEM((TAIL_B,), jnp.int32),
            pltpu.VMEM_SHARED((n, d), jnp.float32),
            pltpu.SemaphoreType.DMA,
            pltpu.SemaphoreType.DMA,
            pltpu.SemaphoreType.DMA,
            pltpu.SemaphoreType.DMA,
        ],
    )
    def kern(h_hbm, src_hbm, dst_hbm, w_hbm, dinv_hbm, out_hbm,
             sv0, sv1, dv0, dv1, w0, w1, rows0, rows1, dinvbuf, tsv, tdv,
             out_sh, g0, g1, s0, s1):
        cid = lax.axis_index("c")
        sid = lax.axis_index("s")
        wid = cid * NS + sid
        svs = (sv0, sv1)
        dvs = (dv0, dv1)
        wvs = (w0, w1)
        rowss = (rows0, rows1)
        gsems = (g0, g1)
        ssems = (s0, s1)

        def zr(i, carry):
            rows0[i // kd, pl.ds((i % kd) * LN, LN)] = jnp.zeros(
                (LN,), jnp.float32)
            return carry
        lax.fori_loop(0, CHUNK_B * kd, zr, 0)
        for t in range((nzc + NS - 1) // NS):
            k = sid + t * NS

            @pl.when(k < nzc)
            def _zc():
                pltpu.sync_copy(rows0, out_sh.at[pl.ds(k * CHUNK_B, CHUNK_B)])

        @pl.when(sid == 0)
        def _zrem():
            pltpu.sync_copy(rows0.at[pl.ds(0, zrem)],
                            out_sh.at[pl.ds(nzc * CHUNK_B, zrem)])
        pltpu.sync_copy(dinv_hbm.at[pl.ds(0, n)], dinvbuf)
        plsc.subcore_barrier()

        base0 = wid * epw

        def load_idx(b, j):
            base = base0 + j * CHUNK_B
            pltpu.sync_copy(src_hbm.at[pl.ds(base, CHUNK_B)], svs[b])
            pltpu.sync_copy(dst_hbm.at[pl.ds(base, CHUNK_B)], dvs[b])
            pltpu.sync_copy(w_hbm.at[pl.ds(base, CHUNK_B)], wvs[b])

        def issue_gather(b):
            pltpu.async_copy(h_hbm.at[svs[b]], rowss[b], gsems[b])

        def wait_gather(b):
            pltpu.make_async_copy(h_hbm.at[svs[b]], rowss[b], gsems[b]).wait()

        def issue_scatter(b):
            pltpu.async_copy(
                rowss[b], out_sh.at[dvs[b]], ssems[b], add=True)

        def wait_scatter(b):
            pltpu.make_async_copy(
                rowss[b], out_sh.at[dvs[b]], ssems[b]).wait()

        def compute(dstv, wv_b, rows_b, ngroups):
            def pv(g, c2):
                sl = pl.ds(g * LN, LN)
                dv16 = dstv[sl]
                wv_b[sl] = wv_b[sl] * plsc.load_gather(dinvbuf, [dv16])
                return c2
            lax.fori_loop(0, ngroups, pv, 0)

            def scale(eidx, c2):
                p = plsc.load_gather(
                    wv_b, [jnp.full((LN,), eidx, jnp.int32)])
                for k in range(kd):
                    sl = pl.ds(k * LN, LN)
                    rows_b[eidx, sl] = rows_b[eidx, sl] * p
                return c2
            lax.fori_loop(0, ngroups * LN, scale, 0)

        load_idx(0, 0)
        issue_gather(0)

        def pair(j2, carry):
            for b in (0, 1):
                j = 2 * j2 + b
                wait_gather(b)

                @pl.when(j >= 1)
                def _():
                    wait_scatter(b ^ 1)

                @pl.when(j + 1 < nchunk)
                def _():
                    load_idx(b ^ 1, j + 1)
                    issue_gather(b ^ 1)
                compute(dvs[b], wvs[b], rowss[b], CHUNK_B // LN)
                issue_scatter(b)
            return carry
        lax.fori_loop(0, (nchunk - 1) // 2, pair, 0)

        wait_gather(0)
        wait_scatter(1)
        tbase = base0 + nchunk * CHUNK_B
        pltpu.sync_copy(src_hbm.at[pl.ds(tbase, TAIL_B)], tsv)
        pltpu.sync_copy(dst_hbm.at[pl.ds(tbase, TAIL_B)], tdv)
        pltpu.sync_copy(w_hbm.at[pl.ds(tbase, TAIL_B)],
                        w1.at[pl.ds(0, TAIL_B)])
        pltpu.async_copy(h_hbm.at[tsv], rows1.at[pl.ds(0, TAIL_B)], g1)
        compute(dv0, w0, rows0, CHUNK_B // LN)
        issue_scatter(0)
        pltpu.make_async_copy(
            h_hbm.at[tsv], rows1.at[pl.ds(0, TAIL_B)], g1).wait()
        compute(tdv, w1, rows1, TAIL_B // LN)
        wait_scatter(0)
        pltpu.sync_copy(rows1.at[pl.ds(0, TAIL_B)], out_sh.at[tdv], add=True)

        plsc.subcore_barrier()
        for t in range((nzc + NS - 1) // NS):
            k = sid + t * NS

            @pl.when(k < nzc)
            def _dump():
                pltpu.sync_copy(out_sh.at[pl.ds(k * CHUNK_B, CHUNK_B)],
                                out_hbm.at[cid, pl.ds(k * CHUNK_B, CHUNK_B)])

        @pl.when(sid == 0)
        def _drem():
            pltpu.sync_copy(out_sh.at[pl.ds(nzc * CHUNK_B, zrem)],
                            out_hbm.at[cid, pl.ds(nzc * CHUNK_B, zrem)])

    return kern


def kernel(x, edge_index, beta):
    n, d = x.shape
    e = edge_index.shape[1]
    src = edge_index[0]
    dst = edge_index[1]
    sc_a = _make_sc_edge_weights(n, e, d)
    sc_b = _make_sc_aggregate(n, e, d)
    h = x
    nh = _tc_normalize(x)
    for i in range(beta.shape[0]):
        beta16 = jnp.full((LN,), beta[i], jnp.float32)
        w, denom = sc_a(nh, src, dst, beta16)
        dinv = _tc_dinv(denom.reshape(NC, NPAD // 128, 128)).reshape(NPAD)
        parts = sc_b(h, src, dst, w, dinv)
        h, nh = _tc_combine(parts)
    return h


# speedup vs baseline: 13.1215x; 1.1815x over previous
import functools

import jax
import jax.numpy as jnp
from jax import lax
from jax.experimental import pallas as pl
from jax.experimental.pallas import tpu as pltpu
from jax.experimental.pallas import tpu_sc as plsc

NC = 2
NS = 16
LN = 16
NW = NC * NS
CHUNK_A = 208
CHUNK_B = 144
TAIL_B = 64
NPAD = 10240


def _norm_body(x_ref, nh_ref):
    x = x_ref[...]
    s = jnp.sum(x * x, axis=1, keepdims=True)
    nh_ref[...] = x / jnp.maximum(jnp.sqrt(s), 1e-12)


def _tc_normalize(x):
    n, d = x.shape
    blk = 1000
    return pl.pallas_call(
        _norm_body,
        grid=(n // blk,),
        in_specs=[pl.BlockSpec((blk, d), lambda i: (i, 0))],
        out_specs=pl.BlockSpec((blk, d), lambda i: (i, 0)),
        out_shape=jax.ShapeDtypeStruct((n, d), jnp.float32),
    )(x)


def _combine_body(parts_ref, h_ref, nh_ref):
    h = jnp.maximum(parts_ref[0] + parts_ref[1], 0.0)
    h_ref[...] = h
    s = jnp.sum(h * h, axis=1, keepdims=True)
    nh_ref[...] = h / jnp.maximum(jnp.sqrt(s), 1e-12)


def _tc_combine(parts):
    _, n, d = parts.shape
    blk = 1000
    return pl.pallas_call(
        _combine_body,
        grid=(n // blk,),
        in_specs=[pl.BlockSpec((2, blk, d), lambda i: (0, i, 0))],
        out_specs=(pl.BlockSpec((blk, d), lambda i: (i, 0)),
                   pl.BlockSpec((blk, d), lambda i: (i, 0))),
        out_shape=(jax.ShapeDtypeStruct((n, d), jnp.float32),
                   jax.ShapeDtypeStruct((n, d), jnp.float32)),
    )(parts)


def _dinv_body(dn_ref, o_ref):
    o_ref[...] = 1.0 / (dn_ref[0] + dn_ref[1])


def _tc_dinv(parts):
    _, r, c = parts.shape
    blk = 8
    return pl.pallas_call(
        _dinv_body,
        grid=(r // blk,),
        in_specs=[pl.BlockSpec((2, blk, c), lambda i: (0, i, 0))],
        out_specs=pl.BlockSpec((blk, c), lambda i: (i, 0)),
        out_shape=jax.ShapeDtypeStruct((r, c), jnp.float32),
    )(parts)


def _sc_mesh():
    return plsc.VectorSubcoreMesh(
        core_axis_name="c", subcore_axis_name="s",
        num_cores=NC, num_subcores=NS)


@functools.lru_cache(maxsize=None)
def _make_sc_edge_weights(n, e, d):
    epw = e // NW
    nchunk = epw // CHUNK_A
    tail = epw - nchunk * CHUNK_A
    assert tail == LN
    kd = d // LN

    @functools.partial(
        pl.kernel,
        out_type=(jax.ShapeDtypeStruct((e,), jnp.float32),
                  jax.ShapeDtypeStruct((NC, NPAD), jnp.float32)),
        mesh=_sc_mesh(),
        compiler_params=pltpu.CompilerParams(needs_layout_passes=False),
        scratch_types=[
            pltpu.VMEM((CHUNK_A,), jnp.int32),
            pltpu.VMEM((CHUNK_A,), jnp.int32),
            pltpu.VMEM((CHUNK_A,), jnp.int32),
            pltpu.VMEM((CHUNK_A,), jnp.int32),
            pltpu.VMEM((CHUNK_A, d), jnp.float32),
            pltpu.VMEM((CHUNK_A, d), jnp.float32),
            pltpu.VMEM((CHUNK_A, d), jnp.float32),
            pltpu.VMEM((CHUNK_A, d), jnp.float32),
            pltpu.VMEM((CHUNK_A,), jnp.float32),
            pltpu.VMEM((CHUNK_A,), jnp.float32),
            pltpu.VMEM((LN * LN,), jnp.float32),
            pltpu.VMEM((LN,), jnp.float32),
            pltpu.VMEM((NPAD,), jnp.float32),
            pltpu.VMEM((2, LN), jnp.int32),
            pltpu.VMEM_SHARED((NPAD,), jnp.float32),
            pltpu.SemaphoreType.DMA,
            pltpu.SemaphoreType.DMA,
            pltpu.SemaphoreType.DMA,
            pltpu.SemaphoreType.DMA,
            pltpu.SemaphoreType.DMA,
            pltpu.SemaphoreType.DMA,
            pltpu.SemaphoreType.DMA,
            pltpu.SemaphoreType.DMA,
            pltpu.SemaphoreType.DMA,
            pltpu.SemaphoreType.DMA,
        ],
    )
    def kern(nh_hbm, src_hbm, dst_hbm, beta_hbm, w_hbm, denom_hbm,
             sv0, sv1, dv0, dv1, sr0, sr1, dr0, dr1, w0, w1, accv, betav,
             nbuf, tiv, denom_sh, gs0, gs1, gd0, gd1, ss0, ss1, ws0, ws1,
             is0, is1):
        cid = lax.axis_index("c")
        sid = lax.axis_index("s")
        wid = cid * NS + sid
        svs = (sv0, sv1)
        dvs = (dv0, dv1)
        srowss = (sr0, sr1)
        drowss = (dr0, dr1)
        wvs = (w0, w1)
        gssems = (gs0, gs1)
        gdsems = (gd0, gd1)
        ssems = (ss0, ss1)
        wsems = (ws0, ws1)

        @pl.when(sid == 0)
        def _zero_denom():
            def zb(i, carry):
                nbuf[pl.ds(i * LN, LN)] = jnp.zeros((LN,), jnp.float32)
                return carry
            lax.fori_loop(0, NPAD // LN, zb, 0)
            pltpu.sync_copy(nbuf, denom_sh)

        pltpu.sync_copy(beta_hbm, betav)
        plsc.subcore_barrier()

        bvec = betav[...]
        absb = jnp.abs(bvec)
        base0 = wid * epw
        lane = lax.broadcasted_iota(jnp.int32, (LN,), 0)

        def load_idx(b, j):
            base = base0 + j * CHUNK_A
            c0 = pltpu.async_copy(src_hbm.at[pl.ds(base, CHUNK_A)],
                                  svs[b], is0)
            c1 = pltpu.async_copy(dst_hbm.at[pl.ds(base, CHUNK_A)],
                                  dvs[b], is1)
            c0.wait()
            c1.wait()

        def issue_gathers(b):
            pltpu.async_copy(nh_hbm.at[svs[b]], srowss[b], gssems[b])
            pltpu.async_copy(nh_hbm.at[dvs[b]], drowss[b], gdsems[b])

        def wait_gathers(b):
            pltpu.make_async_copy(
                nh_hbm.at[svs[b]], srowss[b], gssems[b]).wait()
            pltpu.make_async_copy(
                nh_hbm.at[dvs[b]], drowss[b], gdsems[b]).wait()

        def issue_scatter(b):
            pltpu.async_copy(
                wvs[b], denom_sh.at[dvs[b]], ssems[b], add=True)

        def wait_scatter(b):
            pltpu.make_async_copy(
                wvs[b], denom_sh.at[dvs[b]], ssems[b]).wait()

        def issue_wstore(b, j):
            base = base0 + j * CHUNK_A
            pltpu.async_copy(wvs[b], w_hbm.at[pl.ds(base, CHUNK_A)], wsems[b])

        def wait_wstore(b, j):
            base = base0 + j * CHUNK_A
            pltpu.make_async_copy(
                wvs[b], w_hbm.at[pl.ds(base, CHUNK_A)], wsems[b]).wait()

        def compute(srows, drows, wv_b, ngroups):
            def grp(g, c2):
                for i in range(LN):
                    ei = g * LN + i
                    acc = srows[ei, pl.ds(0, LN)] * drows[ei, pl.ds(0, LN)]
                    for k in range(1, kd):
                        acc = acc + (srows[ei, pl.ds(k * LN, LN)]
                                     * drows[ei, pl.ds(k * LN, LN)])
                    accv[pl.ds(i * LN, LN)] = acc
                ii = lane * LN
                tot0 = plsc.load_gather(accv, [ii])
                tot1 = plsc.load_gather(accv, [ii + 1])
                for l in range(2, LN, 2):
                    tot0 = tot0 + plsc.load_gather(accv, [ii + l])
                    tot1 = tot1 + plsc.load_gather(accv, [ii + l + 1])
                tot = tot0 + tot1
                wv_b[pl.ds(g * LN, LN)] = jnp.exp(bvec * tot - absb)
                return c2
            lax.fori_loop(0, ngroups, grp, 0)

        load_idx(0, 0)
        issue_gathers(0)

        def pair(j2, carry):
            for b in (0, 1):
                j = 2 * j2 + b
                wait_gathers(b)

                @pl.when(j >= 1)
                def _():
                    wait_scatter(b ^ 1)

                @pl.when(j + 1 < nchunk)
                def _():
                    load_idx(b ^ 1, j + 1)
                    issue_gathers(b ^ 1)

                @pl.when(j >= 2)
                def _():
                    wait_wstore(b, j - 2)
                compute(srowss[b], drowss[b], wvs[b], CHUNK_A // LN)
                issue_wstore(b, j)
                issue_scatter(b)
            return carry
        lax.fori_loop(0, nchunk // 2, pair, 0)

        wait_scatter(1)
        wait_wstore(0, nchunk - 2)
        wait_wstore(1, nchunk - 1)
        tbase = base0 + nchunk * CHUNK_A
        pltpu.sync_copy(src_hbm.at[pl.ds(tbase, tail)], tiv.at[0])
        pltpu.sync_copy(dst_hbm.at[pl.ds(tbase, tail)], tiv.at[1])
        tsrc = tiv.at[0]
        tdst = tiv.at[1]
        pltpu.async_copy(
            nh_hbm.at[tsrc], sr0.at[pl.ds(0, tail)], gs0).wait()
        pltpu.async_copy(
            nh_hbm.at[tdst], dr0.at[pl.ds(0, tail)], gd0).wait()
        compute(sr0, dr0, w0, 1)
        pltpu.sync_copy(w0.at[pl.ds(0, tail)], w_hbm.at[pl.ds(tbase, tail)])
        pltpu.sync_copy(w0.at[pl.ds(0, tail)], denom_sh.at[tdst], add=True)

        plsc.subcore_barrier()

        @pl.when(sid == 0)
        def _dump():
            pltpu.sync_copy(denom_sh, nbuf)
            pltpu.sync_copy(nbuf, denom_hbm.at[cid])

    return kern


@functools.lru_cache(maxsize=None)
def _make_sc_aggregate(n, e, d):
    epw = e // NW
    nchunk = epw // CHUNK_B
    assert epw - nchunk * CHUNK_B == TAIL_B and TAIL_B % LN == 0
    kd = d // LN
    nzc = n // CHUNK_B
    zrem = n - nzc * CHUNK_B
    assert zrem % 8 == 0

    @functools.partial(
        pl.kernel,
        out_type=jax.ShapeDtypeStruct((NC, n, d), jnp.float32),
        mesh=_sc_mesh(),
        compiler_params=pltpu.CompilerParams(needs_layout_passes=False),
        scratch_types=[
            pltpu.VMEM((CHUNK_B,), jnp.int32),
            pltpu.VMEM((CHUNK_B,), jnp.int32),
            pltpu.VMEM((CHUNK_B,), jnp.int32),
            pltpu.VMEM((CHUNK_B,), jnp.int32),
            pltpu.VMEM((CHUNK_B,), jnp.float32),
            pltpu.VMEM((CHUNK_B,), jnp.float32),
            pltpu.VMEM((CHUNK_B, d), jnp.float32),
            pltpu.VMEM((CHUNK_B, d), jnp.float32),
            pltpu.VMEM((n,), jnp.float32),
            pltpu.VMEM((TAIL_B,), jnp.int32),
            pltpu.VMEM((TAIL_B,), jnp.int32),
            pltpu.VMEM_SHARED((n, d), jnp.float32),
            pltpu.SemaphoreType.DMA,
            pltpu.SemaphoreType.DMA,
            pltpu.SemaphoreType.DMA,
            pltpu.SemaphoreType.DMA,
            pltpu.SemaphoreType.DMA,
            pltpu.SemaphoreType.DMA,
            pltpu.SemaphoreType.DMA,
        ],
    )
    def kern(h_hbm, src_hbm, dst_hbm, w_hbm, dinv_hbm, out_hbm,
             sv0, sv1, dv0, dv1, w0, w1, rows0, rows1, dinvbuf, tsv, tdv,
             out_sh, g0, g1, s0, s1, is0, is1, is2):
        cid = lax.axis_index("c")
        sid = lax.axis_index("s")
        wid = cid * NS + sid
        svs = (sv0, sv1)
        dvs = (dv0, dv1)
        wvs = (w0, w1)
        rowss = (rows0, rows1)
        gsems = (g0, g1)
        ssems = (s0, s1)

        def zr(i, carry):
            rows0[i // kd, pl.ds((i % kd) * LN, LN)] = jnp.zeros(
                (LN,), jnp.float32)
            return carry
        lax.fori_loop(0, CHUNK_B * kd, zr, 0)
        for t in range((nzc + NS - 1) // NS):
            k = sid + t * NS

            @pl.when(k < nzc)
            def _zc():
                pltpu.sync_copy(rows0, out_sh.at[pl.ds(k * CHUNK_B, CHUNK_B)])

        @pl.when(sid == 0)
        def _zrem():
            pltpu.sync_copy(rows0.at[pl.ds(0, zrem)],
                            out_sh.at[pl.ds(nzc * CHUNK_B, zrem)])
        pltpu.sync_copy(dinv_hbm.at[pl.ds(0, n)], dinvbuf)
        plsc.subcore_barrier()

        base0 = wid * epw

        def load_idx(b, j):
            base = base0 + j * CHUNK_B
            c0 = pltpu.async_copy(src_hbm.at[pl.ds(base, CHUNK_B)],
                                  svs[b], is0)
            c1 = pltpu.async_copy(dst_hbm.at[pl.ds(base, CHUNK_B)],
                                  dvs[b], is1)
            c2 = pltpu.async_copy(w_hbm.at[pl.ds(base, CHUNK_B)],
                                  wvs[b], is2)
            c0.wait()
            c1.wait()
            c2.wait()

        def issue_gather(b):
            pltpu.async_copy(h_hbm.at[svs[b]], rowss[b], gsems[b])

        def wait_gather(b):
            pltpu.make_async_copy(h_hbm.at[svs[b]], rowss[b], gsems[b]).wait()

        def issue_scatter(b):
            pltpu.async_copy(
                rowss[b], out_sh.at[dvs[b]], ssems[b], add=True)

        def wait_scatter(b):
            pltpu.make_async_copy(
                rowss[b], out_sh.at[dvs[b]], ssems[b]).wait()

        def compute(dstv, wv_b, rows_b, ngroups):
            def pv(g, c2):
                sl = pl.ds(g * LN, LN)
                dv16 = dstv[sl]
                wv_b[sl] = wv_b[sl] * plsc.load_gather(dinvbuf, [dv16])
                return c2
            lax.fori_loop(0, ngroups, pv, 0)

            def scale(eidx, c2):
                p = plsc.load_gather(
                    wv_b, [jnp.full((LN,), eidx, jnp.int32)])
                for k in range(kd):
                    sl = pl.ds(k * LN, LN)
                    rows_b[eidx, sl] = rows_b[eidx, sl] * p
                return c2
            lax.fori_loop(0, ngroups * LN, scale, 0)

        load_idx(0, 0)
        issue_gather(0)

        def pair(j2, carry):
            for b in (0, 1):
                j = 2 * j2 + b
                wait_gather(b)

                @pl.when(j >= 1)
                def _():
                    wait_scatter(b ^ 1)

                @pl.when(j + 1 < nchunk)
                def _():
                    load_idx(b ^ 1, j + 1)
                    issue_gather(b ^ 1)
                compute(dvs[b], wvs[b], rowss[b], CHUNK_B // LN)
                issue_scatter(b)
            return carry
        lax.fori_loop(0, (nchunk - 1) // 2, pair, 0)

        wait_gather(0)
        wait_scatter(1)
        tbase = base0 + nchunk * CHUNK_B
        pltpu.sync_copy(src_hbm.at[pl.ds(tbase, TAIL_B)], tsv)
        pltpu.sync_copy(dst_hbm.at[pl.ds(tbase, TAIL_B)], tdv)
        pltpu.sync_copy(w_hbm.at[pl.ds(tbase, TAIL_B)],
                        w1.at[pl.ds(0, TAIL_B)])
        pltpu.async_copy(h_hbm.at[tsv], rows1.at[pl.ds(0, TAIL_B)], g1)
        compute(dv0, w0, rows0, CHUNK_B // LN)
        issue_scatter(0)
        pltpu.make_async_copy(
            h_hbm.at[tsv], rows1.at[pl.ds(0, TAIL_B)], g1).wait()
        compute(tdv, w1, rows1, TAIL_B // LN)
        wait_scatter(0)
        pltpu.sync_copy(rows1.at[pl.ds(0, TAIL_B)], out_sh.at[tdv], add=True)

        plsc.subcore_barrier()
        for t in range((nzc + NS - 1) // NS):
            k = sid + t * NS

            @pl.when(k < nzc)
            def _dump():
                pltpu.sync_copy(out_sh.at[pl.ds(k * CHUNK_B, CHUNK_B)],
                                out_hbm.at[cid, pl.ds(k * CHUNK_B, CHUNK_B)])

        @pl.when(sid == 0)
        def _drem():
            pltpu.sync_copy(out_sh.at[pl.ds(nzc * CHUNK_B, zrem)],
                            out_hbm.at[cid, pl.ds(nzc * CHUNK_B, zrem)])

    return kern


def kernel(x, edge_index, beta):
    n, d = x.shape
    e = edge_index.shape[1]
    src = edge_index[0]
    dst = edge_index[1]
    sc_a = _make_sc_edge_weights(n, e, d)
    sc_b = _make_sc_aggregate(n, e, d)
    h = x
    nh = _tc_normalize(x)
    for i in range(beta.shape[0]):
        beta16 = jnp.full((LN,), beta[i], jnp.float32)
        w, denom = sc_a(nh, src, dst, beta16)
        dinv = _tc_dinv(denom.reshape(NC, NPAD // 128, 128)).reshape(NPAD)
        parts = sc_b(h, src, dst, w, dinv)
        h, nh = _tc_combine(parts)
    return h
